# Initial kernel scaffold; baseline (speedup 1.0000x reference)
import jax
import jax.numpy as jnp
from jax import lax
from jax.experimental import pallas as pl
from jax.experimental.pallas import tpu as pltpu
from jax.experimental.pallas import tpu_sc as plsc

N = 10000
E = 160000
D = 256

NC = 2
NS = 16
L = 16

BATCH = 128
NBG = BATCH // L


def _make_agg(n, e, d, compute_deg, chunk):
    assert d % L == 0
    nd16 = d // L
    es = e // NS
    assert es % chunk == 0 and chunk % L == 0 and es % 8 == 0 and chunk % 8 == 0
    nch = es // chunk
    ngrp = chunk // L
    rpt = ((n + NC * NS * 8 - 1) // (NC * NS * 8)) * 8
    rpc = rpt * NS
    npad = rpc * NC
    trash = rpc
    accr = rpc + 8
    stg = chunk + BATCH + L

    def body(src_hbm, dst_hbm, w_hbm, h_hbm, *rest):
        if compute_deg:
            (out_hbm, deg_hbm, acc_sh, deg_sh, src_ck, dst_ck, w_ck,
             st_src, st_dl, st_w, b_src, b_dl, b_w, gbuf, ones_v) = rest
        else:
            (out_hbm, acc_sh, src_ck, dst_ck, w_ck,
             st_src, st_dl, st_w, b_src, b_dl, b_w, gbuf, ones_v) = rest
        c = lax.axis_index("c")
        s = lax.axis_index("s")
        r0 = s * rpt
        base_row = c * rpc + r0
        sc_lo = c * rpc
        iota = lax.iota(jnp.int32, L)
        zv = jnp.zeros((L,), jnp.float32)

        def zrow(r, _):
            for k in range(nd16):
                gbuf[r, pl.ds(k * L, L)] = zv
            return 0
        lax.fori_loop(0, BATCH, zrow, 0)
        off = 0
        while off < rpt:
            step = min(BATCH, rpt - off)
            pltpu.sync_copy(gbuf.at[pl.ds(0, step)],
                            acc_sh.at[pl.ds(r0 + off, step)])
            off += step
        if compute_deg:
            for g in range(rpt // L):
                st_w[pl.ds(g * L, L)] = zv
            pltpu.sync_copy(st_w.at[pl.ds(0, rpt)], deg_sh.at[pl.ds(r0, rpt)])
        for g in range(NBG):
            ones_v[pl.ds(g * L, L)] = jnp.ones((L,), jnp.float32)
        plsc.subcore_barrier()

        def do_chunk(k, _):
            ebase = s * es + k * chunk
            pltpu.sync_copy(src_hbm.at[pl.ds(ebase, chunk)], src_ck)
            pltpu.sync_copy(dst_hbm.at[pl.ds(ebase, chunk)], dst_ck)
            pltpu.sync_copy(w_hbm.at[pl.ds(ebase, chunk)], w_ck)

            def group(g, ptr):
                d16 = dst_ck[pl.ds(g * L, L)]
                s16 = src_ck[pl.ds(g * L, L)]
                w16 = w_ck[pl.ds(g * L, L)]
                dl = d16 - sc_lo
                m = (dl >= 0) & (dl < rpc)
                plsc.store_compressed(st_src.at[pl.ds(ptr, L)], s16, mask=m)
                plsc.store_compressed(st_dl.at[pl.ds(ptr, L)], dl, mask=m)
                plsc.store_compressed(st_w.at[pl.ds(ptr, L)], w16, mask=m)
                return ptr + jnp.sum(m.astype(jnp.int32))

            ptr = lax.fori_loop(0, ngrp, group, jnp.int32(0))

            def batch(b, _):
                off = b * BATCH
                for g in range(NBG):
                    gi = off + g * L
                    m = (iota + gi) < ptr
                    sv = st_src[pl.ds(gi, L)]
                    dv = st_dl[pl.ds(gi, L)]
                    wv = st_w[pl.ds(gi, L)]
                    b_src[pl.ds(g * L, L)] = jnp.where(m, sv, 0)
                    b_dl[pl.ds(g * L, L)] = jnp.where(m, dv, trash)
                    b_w[pl.ds(g * L, L)] = jnp.where(m, wv, 0.0)
                pltpu.sync_copy(h_hbm.at[b_src], gbuf)

                def srow(r, _):
                    wsp = plsc.load_gather(b_w, [jnp.full((L,), r, jnp.int32)])
                    for k in range(nd16):
                        sl = pl.ds(k * L, L)
                        gbuf[r, sl] = gbuf[r, sl] * wsp
                    return 0
                lax.fori_loop(0, BATCH, srow, 0)

                pltpu.sync_copy(gbuf, acc_sh.at[b_dl], add=True)
                if compute_deg:
                    pltpu.sync_copy(ones_v, deg_sh.at[b_dl], add=True)
                return 0

            nb = (ptr + (BATCH - 1)) // BATCH
            lax.fori_loop(0, nb, batch, 0)
            return 0

        lax.fori_loop(0, nch, do_chunk, 0)
        plsc.subcore_barrier()

        pltpu.sync_copy(acc_sh.at[pl.ds(r0, rpt)],
                        out_hbm.at[pl.ds(base_row, rpt)])
        if compute_deg:
            pltpu.sync_copy(deg_sh.at[pl.ds(r0, rpt)],
                            deg_hbm.at[pl.ds(base_row, rpt)])

    out_type = [jax.ShapeDtypeStruct((npad, d), jnp.float32)]
    if compute_deg:
        out_type.append(jax.ShapeDtypeStruct((npad,), jnp.float32))
    scratch = [pltpu.VMEM_SHARED((accr, d), jnp.float32)]
    if compute_deg:
        scratch.append(pltpu.VMEM_SHARED((accr,), jnp.float32))
    scratch += [
        pltpu.VMEM((chunk,), jnp.int32),
        pltpu.VMEM((chunk,), jnp.int32),
        pltpu.VMEM((chunk,), jnp.float32),
        pltpu.VMEM((stg,), jnp.int32),
        pltpu.VMEM((stg,), jnp.int32),
        pltpu.VMEM((stg,), jnp.float32),
        pltpu.VMEM((BATCH,), jnp.int32),
        pltpu.VMEM((BATCH,), jnp.int32),
        pltpu.VMEM((BATCH,), jnp.float32),
        pltpu.VMEM((BATCH, d), jnp.float32),
        pltpu.VMEM((BATCH,), jnp.float32),
    ]
    mesh = plsc.VectorSubcoreMesh(core_axis_name="c", subcore_axis_name="s",
                                  num_cores=NC, num_subcores=NS)
    return pl.kernel(body, out_type=tuple(out_type), mesh=mesh,
                     scratch_types=scratch), npad


def _dense_layer(h, agg_raw, deg, w_self, w_neigh, b, relu, bm=400):
    n, d = h.shape
    assert n % bm == 0
    dn = (((1,), (1,)), ((), ()))

    def body(x_ref, agg_ref, deg_ref, ws_ref, wn_ref, b_ref, o_ref):
        inv = 1.0 / jnp.maximum(deg_ref[...], 1.0)
        scaled = agg_ref[...] * inv
        acc = lax.dot_general(x_ref[...], ws_ref[...], dn,
                              preferred_element_type=jnp.float32)
        acc = acc + lax.dot_general(scaled, wn_ref[...], dn,
                                    preferred_element_type=jnp.float32)
        acc = acc + b_ref[...]
        o_ref[...] = jnp.maximum(acc, 0.0) if relu else acc

    return pl.pallas_call(
        body,
        grid=(n // bm,),
        in_specs=[
            pl.BlockSpec((bm, d), lambda i: (i, 0)),
            pl.BlockSpec((bm, d), lambda i: (i, 0)),
            pl.BlockSpec((bm, 1), lambda i: (i, 0)),
            pl.BlockSpec((d, d), lambda i: (0, 0)),
            pl.BlockSpec((d, d), lambda i: (0, 0)),
            pl.BlockSpec((1, d), lambda i: (0, 0)),
        ],
        out_specs=pl.BlockSpec((bm, d), lambda i: (i, 0)),
        out_shape=jax.ShapeDtypeStruct((n, d), jnp.float32),
    )(h, agg_raw, deg, w_self, w_neigh, b)


def kernel(x, edge_index, edge_weight, W_self0, W_neigh0, b0,
           W_self1, W_neigh1, b1):
    src = edge_index[0].astype(jnp.int32)
    dst = edge_index[1].astype(jnp.int32)
    w = edge_weight.astype(jnp.float32)
    agg_deg_call, _ = _make_agg(N, E, D, compute_deg=True, chunk=2000)
    agg_call, _ = _make_agg(N, E, D, compute_deg=False, chunk=2000)
    agg1, deg = agg_deg_call(src, dst, w, x)
    deg2d = deg[:N].reshape(N, 1)
    h1 = _dense_layer(x, agg1[:N], deg2d, W_self0, W_neigh0,
                      b0.reshape(1, D), relu=True)
    agg2 = agg_call(src, dst, w, h1)
    out = _dense_layer(h1, agg2[:N], deg2d, W_self1, W_neigh1,
                       b1.reshape(1, D), relu=False)
    return out


# speedup vs baseline: 1.6559x; 1.6559x over previous
import jax
import jax.numpy as jnp
from jax import lax
from jax.experimental import pallas as pl
from jax.experimental.pallas import tpu as pltpu
from jax.experimental.pallas import tpu_sc as plsc

N = 10000
E = 160000
D = 256

NC = 2
NS = 16
NT = NC * NS
L = 16

BATCH = 128
NBG = BATCH // L
LC = 2048
PKS = 512

_SC_PARAMS = pltpu.CompilerParams(needs_layout_passes=False)


def _mesh():
    return plsc.VectorSubcoreMesh(core_axis_name="c", subcore_axis_name="s",
                                  num_cores=NC, num_subcores=NS)


def _derived(n, e):
    rpt = ((n + NT * 8 - 1) // (NT * 8)) * 8
    region = ((e + LC + 127) // 128) * 128 + 128
    return rpt, region


def _make_bucket(n, e, ce, flush):
    rpt, region = _derived(n, e)
    assert rpt < PKS and e % ce == 0 and ce % L == 0 and ce % 8 == 0
    nch = e // ce
    ngrp = ce // L
    stash = ((ce + flush + L + 7) // 8) * 8
    stsz = stash + L

    def body(src_hbm, dst_hbm, w_hbm, pk_out, w_out, cnt_out,
             src_ck, dst_ck, w_ck, st_pk, st_w, cntb):
        c = lax.axis_index("c")
        s = lax.axis_index("s")
        tid = c * NS + s
        lo = tid * rpt
        tbase = tid * region
        iota = lax.iota(jnp.int32, L)

        def chunk(ch, carry):
            ptr, total = carry
            ebase = ch * ce
            pltpu.sync_copy(src_hbm.at[pl.ds(ebase, ce)], src_ck)
            pltpu.sync_copy(dst_hbm.at[pl.ds(ebase, ce)], dst_ck)
            pltpu.sync_copy(w_hbm.at[pl.ds(ebase, ce)], w_ck)

            def group(g, p):
                d16 = dst_ck[pl.ds(g * L, L)]
                s16 = src_ck[pl.ds(g * L, L)]
                w16 = w_ck[pl.ds(g * L, L)]
                dl = d16 - lo
                m = (dl >= 0) & (dl < rpt)
                csum = plsc.cumsum(jnp.where(m, 1, 0))
                pos = jnp.where(m, p + csum - 1, stash + iota)
                plsc.store_scatter(st_pk, [pos], s16 * PKS + dl)
                plsc.store_scatter(st_w, [pos], w16)
                return p + csum[L - 1]

            ptr = lax.fori_loop(0, ngrp, group, ptr)
            nfl = ptr // flush

            def fl(f, _):
                o = f * flush
                dst_off = pl.multiple_of(tbase + total + o, flush)
                pltpu.sync_copy(st_pk.at[pl.ds(o, flush)],
                                pk_out.at[pl.ds(dst_off, flush)])
                pltpu.sync_copy(st_w.at[pl.ds(o, flush)],
                                w_out.at[pl.ds(dst_off, flush)])
                return 0

            lax.fori_loop(0, nfl, fl, 0)
            moved = nfl * flush

            @pl.when(nfl > 0)
            def _tail():
                for g in range(flush // L):
                    sl = pl.ds(g * L, L)
                    st_pk[sl] = st_pk[pl.ds(moved + g * L, L)]
                    st_w[sl] = st_w[pl.ds(moved + g * L, L)]

            return ptr - moved, total + moved

        ptr, total = lax.fori_loop(0, nch, chunk,
                                   (jnp.int32(0), jnp.int32(0)))

        nfin = (ptr + 127) // 128

        def ffin(f, _):
            o = f * 128
            dst_off = pl.multiple_of(tbase + total + o, 128)
            pltpu.sync_copy(st_pk.at[pl.ds(o, 128)],
                            pk_out.at[pl.ds(dst_off, 128)])
            pltpu.sync_copy(st_w.at[pl.ds(o, 128)],
                            w_out.at[pl.ds(dst_off, 128)])
            return 0

        lax.fori_loop(0, nfin, ffin, 0)
        cntb[pl.ds(0, L)] = jnp.full((L,), total + ptr, jnp.int32)
        pltpu.sync_copy(cntb, cnt_out.at[pl.ds(tid * L, L)])

    out_type = (
        jax.ShapeDtypeStruct((NT * region,), jnp.int32),
        jax.ShapeDtypeStruct((NT * region,), jnp.float32),
        jax.ShapeDtypeStruct((NT * L,), jnp.int32),
    )
    scratch = [
        pltpu.VMEM((ce,), jnp.int32),
        pltpu.VMEM((ce,), jnp.int32),
        pltpu.VMEM((ce,), jnp.float32),
        pltpu.VMEM((stsz,), jnp.int32),
        pltpu.VMEM((stsz,), jnp.float32),
        pltpu.VMEM((L,), jnp.int32),
    ]
    return pl.kernel(body, out_type=out_type, mesh=_mesh(),
                     scratch_types=scratch,
                     compiler_params=_SC_PARAMS)


def _make_agg(n, e, d, compute_deg):
    assert d % L == 0
    nd16 = d // L
    rpt, region = _derived(n, e)
    npad = NT * rpt
    trash = rpt
    accw = (rpt + 1) * d
    degsz = rpt + 2 * L

    def body(pk_hbm, wl_hbm, cnt_hbm, h_hbm, *rest):
        if compute_deg:
            (out_hbm, deg_hbm, acc, deg_pad, lst_pk, lst_w,
             b_src, b_dl, b_w, gbuf, cntb) = rest
        else:
            (out_hbm, acc, lst_pk, lst_w,
             b_src, b_dl, b_w, gbuf, cntb) = rest
        c = lax.axis_index("c")
        s = lax.axis_index("s")
        tid = c * NS + s
        tbase = tid * region
        iota = lax.iota(jnp.int32, L)
        zv = jnp.zeros((L,), jnp.float32)
        e0 = jnp.where(iota == 0, 1.0, 0.0)

        def zrow(r, _):
            acc[pl.ds(r * L, L)] = zv
            return 0
        lax.fori_loop(0, accw // L, zrow, 0)
        if compute_deg:
            for g in range(degsz // L):
                deg_pad[pl.ds(g * L, L)] = zv

        pltpu.sync_copy(cnt_hbm.at[pl.ds(tid * L, L)], cntb)
        cnt = cntb[pl.ds(0, L)][0]
        nb = (cnt + (BATCH - 1)) // BATCH
        ncl = (nb + (LC // BATCH - 1)) // (LC // BATCH)

        def list_chunk(ci, _):
            pltpu.sync_copy(pk_hbm.at[pl.ds(tbase + ci * LC, LC)], lst_pk)
            pltpu.sync_copy(wl_hbm.at[pl.ds(tbase + ci * LC, LC)], lst_w)

            def batch(bi, _):
                b = ci * (LC // BATCH) + bi
                for g in range(NBG):
                    off = bi * BATCH + g * L
                    valid = (iota + (b * BATCH + g * L)) < cnt
                    pk = lst_pk[pl.ds(off, L)]
                    wv = lst_w[pl.ds(off, L)]
                    sv = lax.shift_right_logical(pk, 9)
                    dv = lax.bitwise_and(pk, PKS - 1)
                    sl = pl.ds(g * L, L)
                    b_src[sl] = jnp.where(valid, sv, 0)
                    b_dl[sl] = jnp.where(valid, dv, trash)
                    b_w[sl] = jnp.where(valid, wv, 0.0)
                pltpu.sync_copy(h_hbm.at[b_src], gbuf)

                def edge(r, _):
                    rr = jnp.full((L,), r, jnp.int32)
                    wsp = plsc.load_gather(b_w, [rr])
                    dl = plsc.load_gather(b_dl, [rr])[0]
                    dbase = dl * d
                    for k in range(nd16):
                        plsc.addupdate(acc.at[pl.ds(dbase + k * L, L)],
                                       gbuf[r, pl.ds(k * L, L)] * wsp)
                    if compute_deg:
                        plsc.addupdate(deg_pad.at[pl.ds(dl, L)], e0)
                    return 0

                lax.fori_loop(0, BATCH, edge, 0)
                return 0

            nbi = jnp.minimum(nb - ci * (LC // BATCH), LC // BATCH)
            lax.fori_loop(0, nbi, batch, 0)
            return 0

        lax.fori_loop(0, ncl, list_chunk, 0)

        pltpu.sync_copy(acc.at[pl.ds(0, rpt * d)],
                        out_hbm.at[pl.ds(tid * rpt * d, rpt * d)])
        if compute_deg:
            pltpu.sync_copy(deg_pad.at[pl.ds(0, rpt)],
                            deg_hbm.at[pl.ds(tid * rpt, rpt)])

    out_type = [jax.ShapeDtypeStruct((npad * d,), jnp.float32)]
    if compute_deg:
        out_type.append(jax.ShapeDtypeStruct((npad,), jnp.float32))
    scratch = [pltpu.VMEM((accw,), jnp.float32)]
    if compute_deg:
        scratch.append(pltpu.VMEM((degsz,), jnp.float32))
    scratch += [
        pltpu.VMEM((LC,), jnp.int32),
        pltpu.VMEM((LC,), jnp.float32),
        pltpu.VMEM((BATCH,), jnp.int32),
        pltpu.VMEM((BATCH,), jnp.int32),
        pltpu.VMEM((BATCH,), jnp.float32),
        pltpu.VMEM((BATCH, d), jnp.float32),
        pltpu.VMEM((L,), jnp.int32),
    ]
    return pl.kernel(body, out_type=tuple(out_type), mesh=_mesh(),
                     scratch_types=scratch,
                     compiler_params=_SC_PARAMS), npad


def _dense_layer(h, agg_raw, deg, w_self, w_neigh, b, relu, bm=400):
    n, d = h.shape
    assert n % bm == 0
    dn = (((1,), (1,)), ((), ()))

    def body(x_ref, agg_ref, deg_ref, ws_ref, wn_ref, b_ref, o_ref):
        inv = 1.0 / jnp.maximum(deg_ref[...], 1.0)
        scaled = agg_ref[...] * inv
        acc = lax.dot_general(x_ref[...], ws_ref[...], dn,
                              preferred_element_type=jnp.float32)
        acc = acc + lax.dot_general(scaled, wn_ref[...], dn,
                                    preferred_element_type=jnp.float32)
        acc = acc + b_ref[...]
        o_ref[...] = jnp.maximum(acc, 0.0) if relu else acc

    return pl.pallas_call(
        body,
        grid=(n // bm,),
        in_specs=[
            pl.BlockSpec((bm, d), lambda i: (i, 0)),
            pl.BlockSpec((bm, d), lambda i: (i, 0)),
            pl.BlockSpec((bm, 1), lambda i: (i, 0)),
            pl.BlockSpec((d, d), lambda i: (0, 0)),
            pl.BlockSpec((d, d), lambda i: (0, 0)),
            pl.BlockSpec((1, d), lambda i: (0, 0)),
        ],
        out_specs=pl.BlockSpec((bm, d), lambda i: (i, 0)),
        out_shape=jax.ShapeDtypeStruct((n, d), jnp.float32),
    )(h, agg_raw, deg, w_self, w_neigh, b)


def kernel(x, edge_index, edge_weight, W_self0, W_neigh0, b0,
           W_self1, W_neigh1, b1):
    src = edge_index[0].astype(jnp.int32)
    dst = edge_index[1].astype(jnp.int32)
    w = edge_weight.astype(jnp.float32)

    bucket = _make_bucket(N, E, ce=4000, flush=1024)
    agg_deg, npad = _make_agg(N, E, D, compute_deg=True)
    agg, _ = _make_agg(N, E, D, compute_deg=False)

    pk_list, w_list, cnt = bucket(src, dst, w)
    agg1, deg = agg_deg(pk_list, w_list, cnt, x)
    agg1 = agg1.reshape(npad, D)[:N]
    deg2d = deg[:N].reshape(N, 1)
    h1 = _dense_layer(x, agg1, deg2d, W_self0, W_neigh0,
                      b0.reshape(1, D), relu=True)
    (agg2,) = agg(pk_list, w_list, cnt, h1)
    agg2 = agg2.reshape(npad, D)[:N]
    out = _dense_layer(h1, agg2, deg2d, W_self1, W_neigh1,
                       b1.reshape(1, D), relu=False)
    return out


# speedup vs baseline: 2.0621x; 1.2453x over previous
import jax
import jax.numpy as jnp
from jax import lax
from jax.experimental import pallas as pl
from jax.experimental.pallas import tpu as pltpu
from jax.experimental.pallas import tpu_sc as plsc

N = 10000
E = 160000
D = 256

NC = 2
NS = 16
NT = NC * NS
L = 16

BATCH = 64
NBG = BATCH // L
LC = 2048
PKS = 512

_SC_PARAMS = pltpu.CompilerParams(needs_layout_passes=False)


def _mesh():
    return plsc.VectorSubcoreMesh(core_axis_name="c", subcore_axis_name="s",
                                  num_cores=NC, num_subcores=NS)


def _derived(n, e):
    rpt = ((n + NT * 8 - 1) // (NT * 8)) * 8
    region = ((e + LC + 127) // 128) * 128 + 128
    return rpt, region


def _make_bucket(n, e, ce, flush):
    rpt, region = _derived(n, e)
    assert rpt < PKS and e % ce == 0 and ce % L == 0 and ce % 8 == 0
    nch = e // ce
    ngrp = ce // L
    stash = ((ce + flush + L + 7) // 8) * 8
    stsz = stash + L

    assert nch % 2 == 0

    def body(src_hbm, dst_hbm, w_hbm, pk_out, w_out, cnt_out,
             src_ck0, dst_ck0, w_ck0, sem0,
             src_ck1, dst_ck1, w_ck1, sem1, st_pk, st_w, cntb):
        csets = ((src_ck0, dst_ck0, w_ck0, sem0),
                 (src_ck1, dst_ck1, w_ck1, sem1))
        c = lax.axis_index("c")
        s = lax.axis_index("s")
        tid = c * NS + s
        lo = tid * rpt
        tbase = tid * region
        iota = lax.iota(jnp.int32, L)

        def start_load(ch, cset):
            ebase = ch * ce
            pltpu.async_copy(src_hbm.at[pl.ds(ebase, ce)], cset[0], cset[3])
            pltpu.async_copy(dst_hbm.at[pl.ds(ebase, ce)], cset[1], cset[3])
            pltpu.async_copy(w_hbm.at[pl.ds(ebase, ce)], cset[2], cset[3])

        def wait_load(ch, cset):
            ebase = ch * ce
            pltpu.make_async_copy(
                src_hbm.at[pl.ds(ebase, ce)], cset[0], cset[3]).wait()
            pltpu.make_async_copy(
                dst_hbm.at[pl.ds(ebase, ce)], cset[1], cset[3]).wait()
            pltpu.make_async_copy(
                w_hbm.at[pl.ds(ebase, ce)], cset[2], cset[3]).wait()

        start_load(0, csets[0])

        def chunk(ch, carry, src_ck, dst_ck, w_ck):
            ptr, total = carry

            def group(g, p):
                d16 = dst_ck[pl.ds(g * L, L)]
                s16 = src_ck[pl.ds(g * L, L)]
                w16 = w_ck[pl.ds(g * L, L)]
                dl = d16 - lo
                m = (dl >= 0) & (dl < rpt)
                csum = plsc.cumsum(jnp.where(m, 1, 0))
                pos = jnp.where(m, p + csum - 1, stash + iota)
                plsc.store_scatter(st_pk, [pos], s16 * PKS + dl)
                plsc.store_scatter(st_w, [pos], w16)
                return p + csum[L - 1]

            ptr = lax.fori_loop(0, ngrp, group, ptr)
            nfl = ptr // flush

            def fl(f, _):
                o = f * flush
                dst_off = pl.multiple_of(tbase + total + o, flush)
                pltpu.sync_copy(st_pk.at[pl.ds(o, flush)],
                                pk_out.at[pl.ds(dst_off, flush)])
                pltpu.sync_copy(st_w.at[pl.ds(o, flush)],
                                w_out.at[pl.ds(dst_off, flush)])
                return 0

            lax.fori_loop(0, nfl, fl, 0)
            moved = nfl * flush

            @pl.when(nfl > 0)
            def _tail():
                for g in range(flush // L):
                    sl = pl.ds(g * L, L)
                    st_pk[sl] = st_pk[pl.ds(moved + g * L, L)]
                    st_w[sl] = st_w[pl.ds(moved + g * L, L)]

            return ptr - moved, total + moved

        def pair(p, carry):
            for sub in range(2):
                ch = 2 * p + sub
                cur = csets[sub]
                wait_load(ch, cur)

                @pl.when(ch + 1 < nch)
                def _():
                    start_load(ch + 1, csets[1 - sub])

                carry = chunk(ch, carry, cur[0], cur[1], cur[2])
            return carry

        ptr, total = lax.fori_loop(0, nch // 2, pair,
                                   (jnp.int32(0), jnp.int32(0)))

        nfin = (ptr + 127) // 128

        def ffin(f, _):
            o = f * 128
            dst_off = pl.multiple_of(tbase + total + o, 128)
            pltpu.sync_copy(st_pk.at[pl.ds(o, 128)],
                            pk_out.at[pl.ds(dst_off, 128)])
            pltpu.sync_copy(st_w.at[pl.ds(o, 128)],
                            w_out.at[pl.ds(dst_off, 128)])
            return 0

        lax.fori_loop(0, nfin, ffin, 0)
        cntb[pl.ds(0, L)] = jnp.full((L,), total + ptr, jnp.int32)
        pltpu.sync_copy(cntb, cnt_out.at[pl.ds(tid * L, L)])

    out_type = (
        jax.ShapeDtypeStruct((NT * region,), jnp.int32),
        jax.ShapeDtypeStruct((NT * region,), jnp.float32),
        jax.ShapeDtypeStruct((NT * L,), jnp.int32),
    )
    scratch = []
    for _ in range(2):
        scratch += [
            pltpu.VMEM((ce,), jnp.int32),
            pltpu.VMEM((ce,), jnp.int32),
            pltpu.VMEM((ce,), jnp.float32),
            pltpu.SemaphoreType.DMA,
        ]
    scratch += [
        pltpu.VMEM((stsz,), jnp.int32),
        pltpu.VMEM((stsz,), jnp.float32),
        pltpu.VMEM((L,), jnp.int32),
    ]
    return pl.kernel(body, out_type=out_type, mesh=_mesh(),
                     scratch_types=scratch,
                     compiler_params=_SC_PARAMS)


def _make_agg(n, e, d, compute_deg):
    assert d % L == 0
    nd16 = d // L
    rpt, region = _derived(n, e)
    npad = NT * rpt
    trash = rpt
    accw = (rpt + 1) * d
    degsz = rpt + 2 * L

    bpc = LC // BATCH

    def body(pk_hbm, wl_hbm, cnt_hbm, h_hbm, *rest):
        if compute_deg:
            (out_hbm, deg_hbm, acc, deg_pad, lst_pk, lst_w,
             b_src0, b_dl0, b_w0, gbuf0, sem0,
             b_src1, b_dl1, b_w1, gbuf1, sem1, cntb) = rest
        else:
            (out_hbm, acc, lst_pk, lst_w,
             b_src0, b_dl0, b_w0, gbuf0, sem0,
             b_src1, b_dl1, b_w1, gbuf1, sem1, cntb) = rest
        bufs = ((b_src0, b_dl0, b_w0, gbuf0, sem0),
                (b_src1, b_dl1, b_w1, gbuf1, sem1))
        c = lax.axis_index("c")
        s = lax.axis_index("s")
        tid = c * NS + s
        tbase = tid * region
        iota = lax.iota(jnp.int32, L)
        zv = jnp.zeros((L,), jnp.float32)
        e0 = jnp.where(iota == 0, 1.0, 0.0)

        def zrow(r, _):
            acc[pl.ds(r * L, L)] = zv
            return 0
        lax.fori_loop(0, accw // L, zrow, 0)
        if compute_deg:
            for g in range(degsz // L):
                deg_pad[pl.ds(g * L, L)] = zv

        pltpu.sync_copy(cnt_hbm.at[pl.ds(tid * L, L)], cntb)
        cnt = cntb[pl.ds(0, L)][0]
        nb = (cnt + (BATCH - 1)) // BATCH
        ncl = (nb + (bpc - 1)) // (bpc)

        def prep(ci, bi, b_src, b_dl, b_w):
            base = ci * bpc + bi
            for g in range(NBG):
                off = bi * BATCH + g * L
                valid = (iota + (base * BATCH + g * L)) < cnt
                pk = lst_pk[pl.ds(off, L)]
                wv = lst_w[pl.ds(off, L)]
                sv = lax.shift_right_logical(pk, 9)
                dv = lax.bitwise_and(pk, PKS - 1)
                sl = pl.ds(g * L, L)
                b_src[sl] = jnp.where(valid, sv, 0)
                b_dl[sl] = jnp.where(valid, dv, trash)
                b_w[sl] = jnp.where(valid, wv, 0.0)

        def compute(b_dl, b_w, gbuf):
            def edge(r, _):
                rr = jnp.full((L,), r, jnp.int32)
                wsp = plsc.load_gather(b_w, [rr])
                dl = plsc.load_gather(b_dl, [rr])[0]
                dbase = dl * d
                for k in range(nd16):
                    plsc.addupdate(acc.at[pl.ds(dbase + k * L, L)],
                                   gbuf[r, pl.ds(k * L, L)] * wsp)
                if compute_deg:
                    plsc.addupdate(deg_pad.at[pl.ds(dl, L)], e0)
                return 0

            lax.fori_loop(0, BATCH, edge, 0)

        def list_chunk(ci, _):
            pltpu.sync_copy(pk_hbm.at[pl.ds(tbase + ci * LC, LC)], lst_pk)
            pltpu.sync_copy(wl_hbm.at[pl.ds(tbase + ci * LC, LC)], lst_w)
            nbi = jnp.minimum(nb - ci * bpc, bpc)

            @pl.when(nbi > 0)
            def _prime():
                prep(ci, jnp.int32(0), b_src0, b_dl0, b_w0)
                pltpu.async_copy(h_hbm.at[b_src0], gbuf0, sem0)

            def pair(p, _):
                for sub in range(2):
                    bi = 2 * p + sub
                    cur = bufs[sub]
                    nxt = bufs[1 - sub]

                    @pl.when(bi < nbi)
                    def _():
                        pltpu.make_async_copy(
                            h_hbm.at[cur[0]], cur[3], cur[4]).wait()

                        @pl.when(bi + 1 < nbi)
                        def _():
                            prep(ci, bi + 1, nxt[0], nxt[1], nxt[2])
                            pltpu.async_copy(
                                h_hbm.at[nxt[0]], nxt[3], nxt[4])

                        compute(cur[1], cur[2], cur[3])
                return 0

            lax.fori_loop(0, (nbi + 1) // 2, pair, 0)
            return 0

        lax.fori_loop(0, ncl, list_chunk, 0)

        pltpu.sync_copy(acc.at[pl.ds(0, rpt * d)],
                        out_hbm.at[pl.ds(tid * rpt * d, rpt * d)])
        if compute_deg:
            pltpu.sync_copy(deg_pad.at[pl.ds(0, rpt)],
                            deg_hbm.at[pl.ds(tid * rpt, rpt)])

    out_type = [jax.ShapeDtypeStruct((npad * d,), jnp.float32)]
    if compute_deg:
        out_type.append(jax.ShapeDtypeStruct((npad,), jnp.float32))
    scratch = [pltpu.VMEM((accw,), jnp.float32)]
    if compute_deg:
        scratch.append(pltpu.VMEM((degsz,), jnp.float32))
    scratch += [
        pltpu.VMEM((LC,), jnp.int32),
        pltpu.VMEM((LC,), jnp.float32),
    ]
    for _ in range(2):
        scratch += [
            pltpu.VMEM((BATCH,), jnp.int32),
            pltpu.VMEM((BATCH,), jnp.int32),
            pltpu.VMEM((BATCH,), jnp.float32),
            pltpu.VMEM((BATCH, d), jnp.float32),
            pltpu.SemaphoreType.DMA,
        ]
    scratch += [pltpu.VMEM((L,), jnp.int32)]
    return pl.kernel(body, out_type=tuple(out_type), mesh=_mesh(),
                     scratch_types=scratch,
                     compiler_params=_SC_PARAMS), npad


def _dense_layer(h, agg_raw, deg, w_self, w_neigh, b, relu, bm=400):
    n, d = h.shape
    assert n % bm == 0
    dn = (((1,), (1,)), ((), ()))

    def body(x_ref, agg_ref, deg_ref, ws_ref, wn_ref, b_ref, o_ref):
        inv = 1.0 / jnp.maximum(deg_ref[...], 1.0)
        scaled = agg_ref[...] * inv
        acc = lax.dot_general(x_ref[...], ws_ref[...], dn,
                              preferred_element_type=jnp.float32)
        acc = acc + lax.dot_general(scaled, wn_ref[...], dn,
                                    preferred_element_type=jnp.float32)
        acc = acc + b_ref[...]
        o_ref[...] = jnp.maximum(acc, 0.0) if relu else acc

    return pl.pallas_call(
        body,
        grid=(n // bm,),
        in_specs=[
            pl.BlockSpec((bm, d), lambda i: (i, 0)),
            pl.BlockSpec((bm, d), lambda i: (i, 0)),
            pl.BlockSpec((bm, 1), lambda i: (i, 0)),
            pl.BlockSpec((d, d), lambda i: (0, 0)),
            pl.BlockSpec((d, d), lambda i: (0, 0)),
            pl.BlockSpec((1, d), lambda i: (0, 0)),
        ],
        out_specs=pl.BlockSpec((bm, d), lambda i: (i, 0)),
        out_shape=jax.ShapeDtypeStruct((n, d), jnp.float32),
    )(h, agg_raw, deg, w_self, w_neigh, b)


def kernel(x, edge_index, edge_weight, W_self0, W_neigh0, b0,
           W_self1, W_neigh1, b1):
    src = edge_index[0].astype(jnp.int32)
    dst = edge_index[1].astype(jnp.int32)
    w = edge_weight.astype(jnp.float32)

    bucket = _make_bucket(N, E, ce=4000, flush=1024)
    agg_deg, npad = _make_agg(N, E, D, compute_deg=True)
    agg, _ = _make_agg(N, E, D, compute_deg=False)

    pk_list, w_list, cnt = bucket(src, dst, w)
    agg1, deg = agg_deg(pk_list, w_list, cnt, x)
    agg1 = agg1.reshape(npad, D)[:N]
    deg2d = deg[:N].reshape(N, 1)
    h1 = _dense_layer(x, agg1, deg2d, W_self0, W_neigh0,
                      b0.reshape(1, D), relu=True)
    (agg2,) = agg(pk_list, w_list, cnt, h1)
    agg2 = agg2.reshape(npad, D)[:N]
    out = _dense_layer(h1, agg2, deg2d, W_self1, W_neigh1,
                       b1.reshape(1, D), relu=False)
    return out


# speedup vs baseline: 2.1626x; 1.0487x over previous
import jax
import jax.numpy as jnp
from jax import lax
from jax.experimental import pallas as pl
from jax.experimental.pallas import tpu as pltpu
from jax.experimental.pallas import tpu_sc as plsc

N = 10000
E = 160000
D = 256

NC = 2
NS = 16
NT = NC * NS
L = 16

BATCH = 64
NBG = BATCH // L
LC = 2048
PKS = 512

_SC_PARAMS = pltpu.CompilerParams(needs_layout_passes=False)


def _mesh():
    return plsc.VectorSubcoreMesh(core_axis_name="c", subcore_axis_name="s",
                                  num_cores=NC, num_subcores=NS)


def _derived(n, e):
    rpt = ((n + NT * 8 - 1) // (NT * 8)) * 8
    region = ((e + LC + 127) // 128) * 128 + 128
    return rpt, region


def _make_bucket(n, e, ce, flush):
    rpt, region = _derived(n, e)
    assert rpt < PKS and e % ce == 0 and ce % L == 0 and ce % 8 == 0
    nch = e // ce
    ngrp = ce // L
    stash = ((ce + flush + L + 7) // 8) * 8
    stsz = stash + L

    assert nch % 2 == 0

    def body(src_hbm, dst_hbm, w_hbm, pk_out, w_out, cnt_out,
             src_ck0, dst_ck0, w_ck0, sem0,
             src_ck1, dst_ck1, w_ck1, sem1, st_pk, st_w, cntb):
        csets = ((src_ck0, dst_ck0, w_ck0, sem0),
                 (src_ck1, dst_ck1, w_ck1, sem1))
        c = lax.axis_index("c")
        s = lax.axis_index("s")
        tid = c * NS + s
        lo = tid * rpt
        tbase = tid * region
        iota = lax.iota(jnp.int32, L)

        def start_load(ch, cset):
            ebase = ch * ce
            pltpu.async_copy(src_hbm.at[pl.ds(ebase, ce)], cset[0], cset[3])
            pltpu.async_copy(dst_hbm.at[pl.ds(ebase, ce)], cset[1], cset[3])
            pltpu.async_copy(w_hbm.at[pl.ds(ebase, ce)], cset[2], cset[3])

        def wait_load(ch, cset):
            ebase = ch * ce
            pltpu.make_async_copy(
                src_hbm.at[pl.ds(ebase, ce)], cset[0], cset[3]).wait()
            pltpu.make_async_copy(
                dst_hbm.at[pl.ds(ebase, ce)], cset[1], cset[3]).wait()
            pltpu.make_async_copy(
                w_hbm.at[pl.ds(ebase, ce)], cset[2], cset[3]).wait()

        start_load(0, csets[0])

        def chunk(ch, carry, src_ck, dst_ck, w_ck):
            ptr, total = carry

            def group(g, p):
                d16 = dst_ck[pl.ds(g * L, L)]
                s16 = src_ck[pl.ds(g * L, L)]
                w16 = w_ck[pl.ds(g * L, L)]
                dl = d16 - lo
                m = (dl >= 0) & (dl < rpt)
                csum = plsc.cumsum(jnp.where(m, 1, 0))
                pos = jnp.where(m, p + csum - 1, stash + iota)
                plsc.store_scatter(st_pk, [pos], s16 * PKS + dl)
                plsc.store_scatter(st_w, [pos], w16)
                return p + csum[L - 1]

            ptr = lax.fori_loop(0, ngrp, group, ptr)
            nfl = ptr // flush

            def fl(f, _):
                o = f * flush
                dst_off = pl.multiple_of(tbase + total + o, flush)
                pltpu.sync_copy(st_pk.at[pl.ds(o, flush)],
                                pk_out.at[pl.ds(dst_off, flush)])
                pltpu.sync_copy(st_w.at[pl.ds(o, flush)],
                                w_out.at[pl.ds(dst_off, flush)])
                return 0

            lax.fori_loop(0, nfl, fl, 0)
            moved = nfl * flush

            @pl.when(nfl > 0)
            def _tail():
                for g in range(flush // L):
                    sl = pl.ds(g * L, L)
                    st_pk[sl] = st_pk[pl.ds(moved + g * L, L)]
                    st_w[sl] = st_w[pl.ds(moved + g * L, L)]

            return ptr - moved, total + moved

        def pair(p, carry):
            for sub in range(2):
                ch = 2 * p + sub
                cur = csets[sub]
                wait_load(ch, cur)

                @pl.when(ch + 1 < nch)
                def _():
                    start_load(ch + 1, csets[1 - sub])

                carry = chunk(ch, carry, cur[0], cur[1], cur[2])
            return carry

        ptr, total = lax.fori_loop(0, nch // 2, pair,
                                   (jnp.int32(0), jnp.int32(0)))

        nfin = (ptr + 127) // 128

        def ffin(f, _):
            o = f * 128
            dst_off = pl.multiple_of(tbase + total + o, 128)
            pltpu.sync_copy(st_pk.at[pl.ds(o, 128)],
                            pk_out.at[pl.ds(dst_off, 128)])
            pltpu.sync_copy(st_w.at[pl.ds(o, 128)],
                            w_out.at[pl.ds(dst_off, 128)])
            return 0

        lax.fori_loop(0, nfin, ffin, 0)
        cntb[pl.ds(0, L)] = jnp.full((L,), total + ptr, jnp.int32)
        pltpu.sync_copy(cntb, cnt_out.at[pl.ds(tid * L, L)])

    out_type = (
        jax.ShapeDtypeStruct((NT * region,), jnp.int32),
        jax.ShapeDtypeStruct((NT * region,), jnp.float32),
        jax.ShapeDtypeStruct((NT * L,), jnp.int32),
    )
    scratch = []
    for _ in range(2):
        scratch += [
            pltpu.VMEM((ce,), jnp.int32),
            pltpu.VMEM((ce,), jnp.int32),
            pltpu.VMEM((ce,), jnp.float32),
            pltpu.SemaphoreType.DMA,
        ]
    scratch += [
        pltpu.VMEM((stsz,), jnp.int32),
        pltpu.VMEM((stsz,), jnp.float32),
        pltpu.VMEM((L,), jnp.int32),
    ]
    return pl.kernel(body, out_type=out_type, mesh=_mesh(),
                     scratch_types=scratch,
                     compiler_params=_SC_PARAMS)


def _make_agg(n, e, d, compute_deg):
    assert d % L == 0
    nd16 = d // L
    rpt, region = _derived(n, e)
    npad = NT * rpt
    trash = rpt
    stride = d + L if compute_deg else d
    accw = (rpt + 1) * stride

    bpc = LC // BATCH

    def body(pk_hbm, wl_hbm, cnt_hbm, h_hbm, *rest):
        (out_hbm, acc, lst_pk, lst_w,
         b_src0, b_dl0, b_w0, gbuf0, sem0,
         b_src1, b_dl1, b_w1, gbuf1, sem1, cntb) = rest
        bufs = ((b_src0, b_dl0, b_w0, gbuf0, sem0),
                (b_src1, b_dl1, b_w1, gbuf1, sem1))
        c = lax.axis_index("c")
        s = lax.axis_index("s")
        tid = c * NS + s
        tbase = tid * region
        iota = lax.iota(jnp.int32, L)
        zv = jnp.zeros((L,), jnp.float32)
        e0 = jnp.where(iota == 0, 1.0, 0.0)

        def zrow(r, _):
            acc[pl.ds(r * L, L)] = zv
            return 0
        lax.fori_loop(0, accw // L, zrow, 0)

        pltpu.sync_copy(cnt_hbm.at[pl.ds(tid * L, L)], cntb)
        cnt = cntb[pl.ds(0, L)][0]
        nb = (cnt + (BATCH - 1)) // BATCH
        ncl = (nb + (bpc - 1)) // (bpc)

        def prep(ci, bi, b_src, b_dl, b_w):
            base = ci * bpc + bi
            for g in range(NBG):
                off = bi * BATCH + g * L
                valid = (iota + (base * BATCH + g * L)) < cnt
                pk = lst_pk[pl.ds(off, L)]
                wv = lst_w[pl.ds(off, L)]
                sv = lax.shift_right_logical(pk, 9)
                dv = lax.bitwise_and(pk, PKS - 1)
                sl = pl.ds(g * L, L)
                b_src[sl] = jnp.where(valid, sv, 0)
                b_dl[sl] = jnp.where(valid, dv, trash)
                b_w[sl] = jnp.where(valid, wv, 0.0)

        def compute(b_dl, b_w, gbuf):
            def edge(r, _):
                rr = jnp.full((L,), r, jnp.int32)
                wsp = plsc.load_gather(b_w, [rr])
                dlv = plsc.load_gather(b_dl, [rr])
                base = dlv * stride + iota
                for k in range(nd16):
                    plsc.addupdate_scatter(
                        acc, [base + (k * L)],
                        gbuf[r, pl.ds(k * L, L)] * wsp)
                if compute_deg:
                    plsc.addupdate_scatter(acc, [base + d], e0)
                return 0

            lax.fori_loop(0, BATCH, edge, 0)

        def list_chunk(ci, _):
            pltpu.sync_copy(pk_hbm.at[pl.ds(tbase + ci * LC, LC)], lst_pk)
            pltpu.sync_copy(wl_hbm.at[pl.ds(tbase + ci * LC, LC)], lst_w)
            nbi = jnp.minimum(nb - ci * bpc, bpc)

            @pl.when(nbi > 0)
            def _prime():
                prep(ci, jnp.int32(0), b_src0, b_dl0, b_w0)
                pltpu.async_copy(h_hbm.at[b_src0], gbuf0, sem0)

            def pair(p, _):
                for sub in range(2):
                    bi = 2 * p + sub
                    cur = bufs[sub]
                    nxt = bufs[1 - sub]

                    @pl.when(bi < nbi)
                    def _():
                        pltpu.make_async_copy(
                            h_hbm.at[cur[0]], cur[3], cur[4]).wait()

                        @pl.when(bi + 1 < nbi)
                        def _():
                            prep(ci, bi + 1, nxt[0], nxt[1], nxt[2])
                            pltpu.async_copy(
                                h_hbm.at[nxt[0]], nxt[3], nxt[4])

                        compute(cur[1], cur[2], cur[3])
                return 0

            lax.fori_loop(0, (nbi + 1) // 2, pair, 0)
            return 0

        lax.fori_loop(0, ncl, list_chunk, 0)

        pltpu.sync_copy(acc.at[pl.ds(0, rpt * stride)],
                        out_hbm.at[pl.ds(tid * rpt * stride, rpt * stride)])

    out_type = [jax.ShapeDtypeStruct((npad * stride,), jnp.float32)]
    scratch = [pltpu.VMEM((accw,), jnp.float32)]
    scratch += [
        pltpu.VMEM((LC,), jnp.int32),
        pltpu.VMEM((LC,), jnp.float32),
    ]
    for _ in range(2):
        scratch += [
            pltpu.VMEM((BATCH,), jnp.int32),
            pltpu.VMEM((BATCH,), jnp.int32),
            pltpu.VMEM((BATCH,), jnp.float32),
            pltpu.VMEM((BATCH, d), jnp.float32),
            pltpu.SemaphoreType.DMA,
        ]
    scratch += [pltpu.VMEM((L,), jnp.int32)]
    return pl.kernel(body, out_type=tuple(out_type), mesh=_mesh(),
                     scratch_types=scratch,
                     compiler_params=_SC_PARAMS), npad, stride


def _dense_layer(h, agg_raw, deg, w_self, w_neigh, b, relu, bm=400):
    n, d = h.shape
    assert n % bm == 0
    dn = (((1,), (1,)), ((), ()))

    def body(x_ref, agg_ref, deg_ref, ws_ref, wn_ref, b_ref, o_ref):
        inv = 1.0 / jnp.maximum(deg_ref[...], 1.0)
        scaled = agg_ref[...] * inv
        acc = lax.dot_general(x_ref[...], ws_ref[...], dn,
                              preferred_element_type=jnp.float32)
        acc = acc + lax.dot_general(scaled, wn_ref[...], dn,
                                    preferred_element_type=jnp.float32)
        acc = acc + b_ref[...]
        o_ref[...] = jnp.maximum(acc, 0.0) if relu else acc

    return pl.pallas_call(
        body,
        grid=(n // bm,),
        in_specs=[
            pl.BlockSpec((bm, d), lambda i: (i, 0)),
            pl.BlockSpec((bm, d), lambda i: (i, 0)),
            pl.BlockSpec((bm, 1), lambda i: (i, 0)),
            pl.BlockSpec((d, d), lambda i: (0, 0)),
            pl.BlockSpec((d, d), lambda i: (0, 0)),
            pl.BlockSpec((1, d), lambda i: (0, 0)),
        ],
        out_specs=pl.BlockSpec((bm, d), lambda i: (i, 0)),
        out_shape=jax.ShapeDtypeStruct((n, d), jnp.float32),
    )(h, agg_raw, deg, w_self, w_neigh, b)


def kernel(x, edge_index, edge_weight, W_self0, W_neigh0, b0,
           W_self1, W_neigh1, b1):
    src = edge_index[0].astype(jnp.int32)
    dst = edge_index[1].astype(jnp.int32)
    w = edge_weight.astype(jnp.float32)

    bucket = _make_bucket(N, E, ce=4000, flush=1024)
    agg_deg, npad, stride1 = _make_agg(N, E, D, compute_deg=True)
    agg, _, stride2 = _make_agg(N, E, D, compute_deg=False)

    pk_list, w_list, cnt = bucket(src, dst, w)
    (agg1f,) = agg_deg(pk_list, w_list, cnt, x)
    agg1f = agg1f.reshape(npad, stride1)
    agg1 = agg1f[:N, :D]
    deg2d = agg1f[:N, D].reshape(N, 1)
    h1 = _dense_layer(x, agg1, deg2d, W_self0, W_neigh0,
                      b0.reshape(1, D), relu=True)
    (agg2,) = agg(pk_list, w_list, cnt, h1)
    agg2 = agg2.reshape(npad, stride2)[:N, :D]
    out = _dense_layer(h1, agg2, deg2d, W_self1, W_neigh1,
                       b1.reshape(1, D), relu=False)
    return out


# speedup vs baseline: 3.9427x; 1.8231x over previous
import jax
import jax.numpy as jnp
from jax import lax
from jax.experimental import pallas as pl
from jax.experimental.pallas import tpu as pltpu
from jax.experimental.pallas import tpu_sc as plsc

N = 10000
E = 160000
D = 256

NC = 2
NS = 16
NT = NC * NS
L = 16

BATCH = 64
NBG = BATCH // L
LC = 2048
PKS = 512

_SC_PARAMS = pltpu.CompilerParams(needs_layout_passes=False)


def _mesh():
    return plsc.VectorSubcoreMesh(core_axis_name="c", subcore_axis_name="s",
                                  num_cores=NC, num_subcores=NS)


def _derived(n, e):
    rpt = ((n + NT * 8 - 1) // (NT * 8)) * 8
    region = ((e + LC + 127) // 128) * 128 + 128
    return rpt, region


def _make_bucket(n, e, ce, flush):
    rpt, region = _derived(n, e)
    assert rpt < PKS and e % ce == 0 and ce % L == 0 and ce % 8 == 0
    nch = e // ce
    ngrp = ce // L
    stash = ((ce + flush + L + 7) // 8) * 8
    stsz = stash + L

    assert nch % 2 == 0

    def body(src_hbm, dst_hbm, w_hbm, pk_out, w_out, cnt_out,
             src_ck0, dst_ck0, w_ck0, sem0,
             src_ck1, dst_ck1, w_ck1, sem1, st_pk, st_w, cntb):
        csets = ((src_ck0, dst_ck0, w_ck0, sem0),
                 (src_ck1, dst_ck1, w_ck1, sem1))
        c = lax.axis_index("c")
        s = lax.axis_index("s")
        tid = c * NS + s
        lo = tid * rpt
        tbase = tid * region
        iota = lax.iota(jnp.int32, L)

        def start_load(ch, cset):
            ebase = ch * ce
            pltpu.async_copy(src_hbm.at[pl.ds(ebase, ce)], cset[0], cset[3])
            pltpu.async_copy(dst_hbm.at[pl.ds(ebase, ce)], cset[1], cset[3])
            pltpu.async_copy(w_hbm.at[pl.ds(ebase, ce)], cset[2], cset[3])

        def wait_load(ch, cset):
            ebase = ch * ce
            pltpu.make_async_copy(
                src_hbm.at[pl.ds(ebase, ce)], cset[0], cset[3]).wait()
            pltpu.make_async_copy(
                dst_hbm.at[pl.ds(ebase, ce)], cset[1], cset[3]).wait()
            pltpu.make_async_copy(
                w_hbm.at[pl.ds(ebase, ce)], cset[2], cset[3]).wait()

        start_load(0, csets[0])

        def chunk(ch, carry, src_ck, dst_ck, w_ck):
            ptr, total = carry

            def group(g, p):
                d16 = dst_ck[pl.ds(g * L, L)]
                s16 = src_ck[pl.ds(g * L, L)]
                w16 = w_ck[pl.ds(g * L, L)]
                dl = d16 - lo
                m = (dl >= 0) & (dl < rpt)
                csum = plsc.cumsum(jnp.where(m, 1, 0))
                pos = jnp.where(m, p + csum - 1, stash + iota)
                plsc.store_scatter(st_pk, [pos], s16 * PKS + dl)
                plsc.store_scatter(st_w, [pos], w16)
                return p + csum[L - 1]

            ptr = lax.fori_loop(0, ngrp, group, ptr)
            nfl = ptr // flush

            def fl(f, _):
                o = f * flush
                dst_off = pl.multiple_of(tbase + total + o, flush)
                pltpu.sync_copy(st_pk.at[pl.ds(o, flush)],
                                pk_out.at[pl.ds(dst_off, flush)])
                pltpu.sync_copy(st_w.at[pl.ds(o, flush)],
                                w_out.at[pl.ds(dst_off, flush)])
                return 0

            lax.fori_loop(0, nfl, fl, 0)
            moved = nfl * flush

            @pl.when(nfl > 0)
            def _tail():
                for g in range(flush // L):
                    sl = pl.ds(g * L, L)
                    st_pk[sl] = st_pk[pl.ds(moved + g * L, L)]
                    st_w[sl] = st_w[pl.ds(moved + g * L, L)]

            return ptr - moved, total + moved

        def pair(p, carry):
            for sub in range(2):
                ch = 2 * p + sub
                cur = csets[sub]
                wait_load(ch, cur)

                @pl.when(ch + 1 < nch)
                def _():
                    start_load(ch + 1, csets[1 - sub])

                carry = chunk(ch, carry, cur[0], cur[1], cur[2])
            return carry

        ptr, total = lax.fori_loop(0, nch // 2, pair,
                                   (jnp.int32(0), jnp.int32(0)))

        nfin = (ptr + 127) // 128

        def ffin(f, _):
            o = f * 128
            dst_off = pl.multiple_of(tbase + total + o, 128)
            pltpu.sync_copy(st_pk.at[pl.ds(o, 128)],
                            pk_out.at[pl.ds(dst_off, 128)])
            pltpu.sync_copy(st_w.at[pl.ds(o, 128)],
                            w_out.at[pl.ds(dst_off, 128)])
            return 0

        lax.fori_loop(0, nfin, ffin, 0)
        cntb[pl.ds(0, L)] = jnp.full((L,), total + ptr, jnp.int32)
        pltpu.sync_copy(cntb, cnt_out.at[pl.ds(tid * L, L)])

    out_type = (
        jax.ShapeDtypeStruct((NT * region,), jnp.int32),
        jax.ShapeDtypeStruct((NT * region,), jnp.float32),
        jax.ShapeDtypeStruct((NT * L,), jnp.int32),
    )
    scratch = []
    for _ in range(2):
        scratch += [
            pltpu.VMEM((ce,), jnp.int32),
            pltpu.VMEM((ce,), jnp.int32),
            pltpu.VMEM((ce,), jnp.float32),
            pltpu.SemaphoreType.DMA,
        ]
    scratch += [
        pltpu.VMEM((stsz,), jnp.int32),
        pltpu.VMEM((stsz,), jnp.float32),
        pltpu.VMEM((L,), jnp.int32),
    ]
    return pl.kernel(body, out_type=out_type, mesh=_mesh(),
                     scratch_types=scratch,
                     compiler_params=_SC_PARAMS)


def _make_agg(n, e, d, compute_deg):
    assert d % L == 0
    nd16 = d // L
    rpt, region = _derived(n, e)
    npad = NT * rpt
    trash = rpt
    stride = d + L if compute_deg else d
    accw = (rpt + 1) * stride

    bpc = LC // BATCH

    def body(pk_hbm, wl_hbm, cnt_hbm, h_hbm, *rest):
        (out_hbm, acc, lst_pk, lst_w,
         b_src0, b_dl0, b_w0, gbuf0, sem0,
         b_src1, b_dl1, b_w1, gbuf1, sem1, cntb) = rest
        bufs = ((b_src0, b_dl0, b_w0, gbuf0, sem0),
                (b_src1, b_dl1, b_w1, gbuf1, sem1))
        c = lax.axis_index("c")
        s = lax.axis_index("s")
        tid = c * NS + s
        tbase = tid * region
        iota = lax.iota(jnp.int32, L)
        zv = jnp.zeros((L,), jnp.float32)
        e0 = jnp.where(iota == 0, 1.0, 0.0)

        def zrow(r, _):
            acc[pl.ds(r * L, L)] = zv
            return 0
        lax.fori_loop(0, accw // L, zrow, 0)

        pltpu.sync_copy(cnt_hbm.at[pl.ds(tid * L, L)], cntb)
        cnt = cntb[pl.ds(0, L)][0]
        nb = (cnt + (BATCH - 1)) // BATCH
        ncl = (nb + (bpc - 1)) // (bpc)

        def prep(ci, bi, b_src, b_dl, b_w):
            base = ci * bpc + bi
            for g in range(NBG):
                off = bi * BATCH + g * L
                valid = (iota + (base * BATCH + g * L)) < cnt
                pk = lst_pk[pl.ds(off, L)]
                wv = lst_w[pl.ds(off, L)]
                sv = lax.shift_right_logical(pk, 9)
                dv = lax.bitwise_and(pk, PKS - 1)
                sl = pl.ds(g * L, L)
                b_src[sl] = jnp.where(valid, sv, 0)
                b_dl[sl] = jnp.where(valid, dv, trash)
                b_w[sl] = jnp.where(valid, wv, 0.0)

        def compute(b_dl, b_w, gbuf):
            @plsc.parallel_loop(0, BATCH, 1, unroll=4)
            def _edges(r):
                rr = jnp.full((L,), r, jnp.int32)
                wsp = plsc.load_gather(b_w, [rr])
                dlv = plsc.load_gather(b_dl, [rr])
                base = dlv * stride + iota
                for k in range(nd16):
                    plsc.addupdate_scatter(
                        acc, [base + (k * L)],
                        gbuf[r, pl.ds(k * L, L)] * wsp)
                if compute_deg:
                    plsc.addupdate_scatter(acc, [base + d], e0)

        def list_chunk(ci, _):
            pltpu.sync_copy(pk_hbm.at[pl.ds(tbase + ci * LC, LC)], lst_pk)
            pltpu.sync_copy(wl_hbm.at[pl.ds(tbase + ci * LC, LC)], lst_w)
            nbi = jnp.minimum(nb - ci * bpc, bpc)

            @pl.when(nbi > 0)
            def _prime():
                prep(ci, jnp.int32(0), b_src0, b_dl0, b_w0)
                pltpu.async_copy(h_hbm.at[b_src0], gbuf0, sem0)

            def pair(p, _):
                for sub in range(2):
                    bi = 2 * p + sub
                    cur = bufs[sub]
                    nxt = bufs[1 - sub]

                    @pl.when(bi < nbi)
                    def _():
                        pltpu.make_async_copy(
                            h_hbm.at[cur[0]], cur[3], cur[4]).wait()

                        @pl.when(bi + 1 < nbi)
                        def _():
                            prep(ci, bi + 1, nxt[0], nxt[1], nxt[2])
                            pltpu.async_copy(
                                h_hbm.at[nxt[0]], nxt[3], nxt[4])

                        compute(cur[1], cur[2], cur[3])
                return 0

            lax.fori_loop(0, (nbi + 1) // 2, pair, 0)
            return 0

        lax.fori_loop(0, ncl, list_chunk, 0)

        pltpu.sync_copy(acc.at[pl.ds(0, rpt * stride)],
                        out_hbm.at[pl.ds(tid * rpt * stride, rpt * stride)])

    out_type = [jax.ShapeDtypeStruct((npad * stride,), jnp.float32)]
    scratch = [pltpu.VMEM((accw,), jnp.float32)]
    scratch += [
        pltpu.VMEM((LC,), jnp.int32),
        pltpu.VMEM((LC,), jnp.float32),
    ]
    for _ in range(2):
        scratch += [
            pltpu.VMEM((BATCH,), jnp.int32),
            pltpu.VMEM((BATCH,), jnp.int32),
            pltpu.VMEM((BATCH,), jnp.float32),
            pltpu.VMEM((BATCH, d), jnp.float32),
            pltpu.SemaphoreType.DMA,
        ]
    scratch += [pltpu.VMEM((L,), jnp.int32)]
    return pl.kernel(body, out_type=tuple(out_type), mesh=_mesh(),
                     scratch_types=scratch,
                     compiler_params=_SC_PARAMS), npad, stride


def _dense_layer(h, agg_raw, deg, w_self, w_neigh, b, relu, bm=400):
    n, d = h.shape
    assert n % bm == 0
    dn = (((1,), (1,)), ((), ()))

    def body(x_ref, agg_ref, deg_ref, ws_ref, wn_ref, b_ref, o_ref):
        inv = 1.0 / jnp.maximum(deg_ref[...], 1.0)
        scaled = agg_ref[...] * inv
        acc = lax.dot_general(x_ref[...], ws_ref[...], dn,
                              preferred_element_type=jnp.float32)
        acc = acc + lax.dot_general(scaled, wn_ref[...], dn,
                                    preferred_element_type=jnp.float32)
        acc = acc + b_ref[...]
        o_ref[...] = jnp.maximum(acc, 0.0) if relu else acc

    return pl.pallas_call(
        body,
        grid=(n // bm,),
        in_specs=[
            pl.BlockSpec((bm, d), lambda i: (i, 0)),
            pl.BlockSpec((bm, d), lambda i: (i, 0)),
            pl.BlockSpec((bm, 1), lambda i: (i, 0)),
            pl.BlockSpec((d, d), lambda i: (0, 0)),
            pl.BlockSpec((d, d), lambda i: (0, 0)),
            pl.BlockSpec((1, d), lambda i: (0, 0)),
        ],
        out_specs=pl.BlockSpec((bm, d), lambda i: (i, 0)),
        out_shape=jax.ShapeDtypeStruct((n, d), jnp.float32),
    )(h, agg_raw, deg, w_self, w_neigh, b)


def kernel(x, edge_index, edge_weight, W_self0, W_neigh0, b0,
           W_self1, W_neigh1, b1):
    src = edge_index[0].astype(jnp.int32)
    dst = edge_index[1].astype(jnp.int32)
    w = edge_weight.astype(jnp.float32)

    bucket = _make_bucket(N, E, ce=4000, flush=1024)
    agg_deg, npad, stride1 = _make_agg(N, E, D, compute_deg=True)
    agg, _, stride2 = _make_agg(N, E, D, compute_deg=False)

    pk_list, w_list, cnt = bucket(src, dst, w)
    (agg1f,) = agg_deg(pk_list, w_list, cnt, x)
    agg1f = agg1f.reshape(npad, stride1)
    agg1 = agg1f[:N, :D]
    deg2d = agg1f[:N, D].reshape(N, 1)
    h1 = _dense_layer(x, agg1, deg2d, W_self0, W_neigh0,
                      b0.reshape(1, D), relu=True)
    (agg2,) = agg(pk_list, w_list, cnt, h1)
    agg2 = agg2.reshape(npad, stride2)[:N, :D]
    out = _dense_layer(h1, agg2, deg2d, W_self1, W_neigh1,
                       b1.reshape(1, D), relu=False)
    return out


# speedup vs baseline: 4.0751x; 1.0336x over previous
import jax
import jax.numpy as jnp
from jax import lax
from jax.experimental import pallas as pl
from jax.experimental.pallas import tpu as pltpu
from jax.experimental.pallas import tpu_sc as plsc

N = 10000
E = 160000
D = 256

NC = 2
NS = 16
NT = NC * NS
L = 16

BATCH = 64
NBG = BATCH // L
LC = 2048
PKS = 512

_SC_PARAMS = pltpu.CompilerParams(needs_layout_passes=False)


def _mesh():
    return plsc.VectorSubcoreMesh(core_axis_name="c", subcore_axis_name="s",
                                  num_cores=NC, num_subcores=NS)


def _derived(n, e):
    rpt = ((n + NT * 8 - 1) // (NT * 8)) * 8
    region = ((e + LC + 127) // 128) * 128 + 128
    return rpt, region


def _make_bucket(n, e, ce, flush):
    rpt, region = _derived(n, e)
    assert rpt < PKS and e % ce == 0 and ce % L == 0 and ce % 8 == 0
    nch = e // ce
    ngrp = ce // L
    stash = ((ce + flush + L + 7) // 8) * 8
    stsz = stash + L

    assert nch % 2 == 0

    def body(src_hbm, dst_hbm, w_hbm, pk_out, w_out, cnt_out,
             src_ck0, dst_ck0, w_ck0, sem0,
             src_ck1, dst_ck1, w_ck1, sem1, st_pk, st_w, cnts, offs, cntb):
        csets = ((src_ck0, dst_ck0, w_ck0, sem0),
                 (src_ck1, dst_ck1, w_ck1, sem1))
        c = lax.axis_index("c")
        s = lax.axis_index("s")
        tid = c * NS + s
        lo = tid * rpt
        tbase = tid * region
        iota = lax.iota(jnp.int32, L)

        def start_load(ch, cset):
            ebase = ch * ce
            pltpu.async_copy(src_hbm.at[pl.ds(ebase, ce)], cset[0], cset[3])
            pltpu.async_copy(dst_hbm.at[pl.ds(ebase, ce)], cset[1], cset[3])
            pltpu.async_copy(w_hbm.at[pl.ds(ebase, ce)], cset[2], cset[3])

        def wait_load(ch, cset):
            ebase = ch * ce
            pltpu.make_async_copy(
                src_hbm.at[pl.ds(ebase, ce)], cset[0], cset[3]).wait()
            pltpu.make_async_copy(
                dst_hbm.at[pl.ds(ebase, ce)], cset[1], cset[3]).wait()
            pltpu.make_async_copy(
                w_hbm.at[pl.ds(ebase, ce)], cset[2], cset[3]).wait()

        start_load(0, csets[0])

        def chunk(ch, carry, src_ck, dst_ck, w_ck):
            ptr, total = carry
            ngrp16 = (ngrp + L - 1) // L
            cnts[pl.ds((ngrp16 - 1) * L, L)] = jnp.zeros((L,), jnp.int32)

            @plsc.parallel_loop(0, ngrp, 1, unroll=4)
            def _count(g):
                d16 = dst_ck[pl.ds(g * L, L)]
                dl = d16 - lo
                m = (dl >= 0) & (dl < rpt)
                cnt = plsc.all_reduce_population_count(m)
                plsc.store_scatter(cnts, [jnp.full((L,), g, jnp.int32)], cnt)

            def pre(i, p):
                c16 = cnts[pl.ds(i * L, L)]
                cs = plsc.cumsum(c16)
                offs[pl.ds(i * L, L)] = p + cs - c16
                return p + cs[L - 1]

            new_ptr = lax.fori_loop(0, ngrp16, pre, ptr)

            @plsc.parallel_loop(0, ngrp, 1, unroll=4)
            def _scatter(g):
                d16 = dst_ck[pl.ds(g * L, L)]
                s16 = src_ck[pl.ds(g * L, L)]
                w16 = w_ck[pl.ds(g * L, L)]
                dl = d16 - lo
                m = (dl >= 0) & (dl < rpt)
                csum = plsc.cumsum(jnp.where(m, 1, 0))
                off = plsc.load_gather(offs, [jnp.full((L,), g, jnp.int32)])
                pos = jnp.where(m, off + csum - 1, stash + iota)
                plsc.store_scatter(st_pk, [pos], s16 * PKS + dl)
                plsc.store_scatter(st_w, [pos], w16)

            ptr = new_ptr
            nfl = ptr // flush

            def fl(f, _):
                o = f * flush
                dst_off = pl.multiple_of(tbase + total + o, flush)
                pltpu.sync_copy(st_pk.at[pl.ds(o, flush)],
                                pk_out.at[pl.ds(dst_off, flush)])
                pltpu.sync_copy(st_w.at[pl.ds(o, flush)],
                                w_out.at[pl.ds(dst_off, flush)])
                return 0

            lax.fori_loop(0, nfl, fl, 0)
            moved = nfl * flush

            @pl.when(nfl > 0)
            def _tail():
                for g in range(flush // L):
                    sl = pl.ds(g * L, L)
                    st_pk[sl] = st_pk[pl.ds(moved + g * L, L)]
                    st_w[sl] = st_w[pl.ds(moved + g * L, L)]

            return ptr - moved, total + moved

        def pair(p, carry):
            for sub in range(2):
                ch = 2 * p + sub
                cur = csets[sub]
                wait_load(ch, cur)

                @pl.when(ch + 1 < nch)
                def _():
                    start_load(ch + 1, csets[1 - sub])

                carry = chunk(ch, carry, cur[0], cur[1], cur[2])
            return carry

        ptr, total = lax.fori_loop(0, nch // 2, pair,
                                   (jnp.int32(0), jnp.int32(0)))

        nfin = (ptr + 127) // 128

        def ffin(f, _):
            o = f * 128
            dst_off = pl.multiple_of(tbase + total + o, 128)
            pltpu.sync_copy(st_pk.at[pl.ds(o, 128)],
                            pk_out.at[pl.ds(dst_off, 128)])
            pltpu.sync_copy(st_w.at[pl.ds(o, 128)],
                            w_out.at[pl.ds(dst_off, 128)])
            return 0

        lax.fori_loop(0, nfin, ffin, 0)
        cntb[pl.ds(0, L)] = jnp.full((L,), total + ptr, jnp.int32)
        pltpu.sync_copy(cntb, cnt_out.at[pl.ds(tid * L, L)])

    out_type = (
        jax.ShapeDtypeStruct((NT * region,), jnp.int32),
        jax.ShapeDtypeStruct((NT * region,), jnp.float32),
        jax.ShapeDtypeStruct((NT * L,), jnp.int32),
    )
    scratch = []
    for _ in range(2):
        scratch += [
            pltpu.VMEM((ce,), jnp.int32),
            pltpu.VMEM((ce,), jnp.int32),
            pltpu.VMEM((ce,), jnp.float32),
            pltpu.SemaphoreType.DMA,
        ]
    ngrp16 = (ngrp + L - 1) // L
    scratch += [
        pltpu.VMEM((stsz,), jnp.int32),
        pltpu.VMEM((stsz,), jnp.float32),
        pltpu.VMEM((ngrp16 * L,), jnp.int32),
        pltpu.VMEM((ngrp16 * L,), jnp.int32),
        pltpu.VMEM((L,), jnp.int32),
    ]
    return pl.kernel(body, out_type=out_type, mesh=_mesh(),
                     scratch_types=scratch,
                     compiler_params=_SC_PARAMS)


def _make_agg(n, e, d, compute_deg):
    assert d % L == 0
    nd16 = d // L
    rpt, region = _derived(n, e)
    npad = NT * rpt
    trash = rpt
    stride = d + L if compute_deg else d
    accw = (rpt + 1) * stride

    bpc = LC // BATCH

    def body(pk_hbm, wl_hbm, cnt_hbm, h_hbm, *rest):
        (out_hbm, acc, lst_pk, lst_w,
         b_src0, b_dl0, b_w0, gbuf0, sem0,
         b_src1, b_dl1, b_w1, gbuf1, sem1, cntb) = rest
        bufs = ((b_src0, b_dl0, b_w0, gbuf0, sem0),
                (b_src1, b_dl1, b_w1, gbuf1, sem1))
        c = lax.axis_index("c")
        s = lax.axis_index("s")
        tid = c * NS + s
        tbase = tid * region
        iota = lax.iota(jnp.int32, L)
        zv = jnp.zeros((L,), jnp.float32)
        e0 = jnp.where(iota == 0, 1.0, 0.0)

        def zrow(r, _):
            acc[pl.ds(r * L, L)] = zv
            return 0
        lax.fori_loop(0, accw // L, zrow, 0)

        pltpu.sync_copy(cnt_hbm.at[pl.ds(tid * L, L)], cntb)
        cnt = cntb[pl.ds(0, L)][0]
        nb = (cnt + (BATCH - 1)) // BATCH
        ncl = (nb + (bpc - 1)) // (bpc)

        def prep(ci, bi, b_src, b_dl, b_w):
            base = ci * bpc + bi
            for g in range(NBG):
                off = bi * BATCH + g * L
                valid = (iota + (base * BATCH + g * L)) < cnt
                pk = lst_pk[pl.ds(off, L)]
                wv = lst_w[pl.ds(off, L)]
                sv = lax.shift_right_logical(pk, 9)
                dv = lax.bitwise_and(pk, PKS - 1)
                sl = pl.ds(g * L, L)
                b_src[sl] = jnp.where(valid, sv, 0)
                b_dl[sl] = jnp.where(valid, dv, trash)
                b_w[sl] = jnp.where(valid, wv, 0.0)

        def compute(b_dl, b_w, gbuf):
            @plsc.parallel_loop(0, BATCH, 1, unroll=8)
            def _edges(r):
                rr = jnp.full((L,), r, jnp.int32)
                wsp = plsc.load_gather(b_w, [rr])
                dlv = plsc.load_gather(b_dl, [rr])
                base = dlv * stride + iota
                for k in range(nd16):
                    plsc.addupdate_scatter(
                        acc, [base + (k * L)],
                        gbuf[r, pl.ds(k * L, L)] * wsp)
                if compute_deg:
                    plsc.addupdate_scatter(acc, [base + d], e0)

        def list_chunk(ci, _):
            pltpu.sync_copy(pk_hbm.at[pl.ds(tbase + ci * LC, LC)], lst_pk)
            pltpu.sync_copy(wl_hbm.at[pl.ds(tbase + ci * LC, LC)], lst_w)
            nbi = jnp.minimum(nb - ci * bpc, bpc)

            @pl.when(nbi > 0)
            def _prime():
                prep(ci, jnp.int32(0), b_src0, b_dl0, b_w0)
                pltpu.async_copy(h_hbm.at[b_src0], gbuf0, sem0)

            def pair(p, _):
                for sub in range(2):
                    bi = 2 * p + sub
                    cur = bufs[sub]
                    nxt = bufs[1 - sub]

                    @pl.when(bi < nbi)
                    def _():
                        pltpu.make_async_copy(
                            h_hbm.at[cur[0]], cur[3], cur[4]).wait()

                        @pl.when(bi + 1 < nbi)
                        def _():
                            prep(ci, bi + 1, nxt[0], nxt[1], nxt[2])
                            pltpu.async_copy(
                                h_hbm.at[nxt[0]], nxt[3], nxt[4])

                        compute(cur[1], cur[2], cur[3])
                return 0

            lax.fori_loop(0, (nbi + 1) // 2, pair, 0)
            return 0

        lax.fori_loop(0, ncl, list_chunk, 0)

        pltpu.sync_copy(acc.at[pl.ds(0, rpt * stride)],
                        out_hbm.at[pl.ds(tid * rpt * stride, rpt * stride)])

    out_type = [jax.ShapeDtypeStruct((npad * stride,), jnp.float32)]
    scratch = [pltpu.VMEM((accw,), jnp.float32)]
    scratch += [
        pltpu.VMEM((LC,), jnp.int32),
        pltpu.VMEM((LC,), jnp.float32),
    ]
    for _ in range(2):
        scratch += [
            pltpu.VMEM((BATCH,), jnp.int32),
            pltpu.VMEM((BATCH,), jnp.int32),
            pltpu.VMEM((BATCH,), jnp.float32),
            pltpu.VMEM((BATCH, d), jnp.float32),
            pltpu.SemaphoreType.DMA,
        ]
    scratch += [pltpu.VMEM((L,), jnp.int32)]
    return pl.kernel(body, out_type=tuple(out_type), mesh=_mesh(),
                     scratch_types=scratch,
                     compiler_params=_SC_PARAMS), npad, stride


def _dense_layer(h, agg_raw, deg, w_self, w_neigh, b, relu, bm=400):
    n, d = h.shape
    assert n % bm == 0
    dn = (((1,), (1,)), ((), ()))

    def body(x_ref, agg_ref, deg_ref, ws_ref, wn_ref, b_ref, o_ref):
        inv = 1.0 / jnp.maximum(deg_ref[...], 1.0)
        scaled = agg_ref[...] * inv
        acc = lax.dot_general(x_ref[...], ws_ref[...], dn,
                              preferred_element_type=jnp.float32)
        acc = acc + lax.dot_general(scaled, wn_ref[...], dn,
                                    preferred_element_type=jnp.float32)
        acc = acc + b_ref[...]
        o_ref[...] = jnp.maximum(acc, 0.0) if relu else acc

    return pl.pallas_call(
        body,
        grid=(n // bm,),
        in_specs=[
            pl.BlockSpec((bm, d), lambda i: (i, 0)),
            pl.BlockSpec((bm, d), lambda i: (i, 0)),
            pl.BlockSpec((bm, 1), lambda i: (i, 0)),
            pl.BlockSpec((d, d), lambda i: (0, 0)),
            pl.BlockSpec((d, d), lambda i: (0, 0)),
            pl.BlockSpec((1, d), lambda i: (0, 0)),
        ],
        out_specs=pl.BlockSpec((bm, d), lambda i: (i, 0)),
        out_shape=jax.ShapeDtypeStruct((n, d), jnp.float32),
    )(h, agg_raw, deg, w_self, w_neigh, b)


def kernel(x, edge_index, edge_weight, W_self0, W_neigh0, b0,
           W_self1, W_neigh1, b1):
    src = edge_index[0].astype(jnp.int32)
    dst = edge_index[1].astype(jnp.int32)
    w = edge_weight.astype(jnp.float32)

    bucket = _make_bucket(N, E, ce=4000, flush=1024)
    agg_deg, npad, stride1 = _make_agg(N, E, D, compute_deg=True)
    agg, _, stride2 = _make_agg(N, E, D, compute_deg=False)

    pk_list, w_list, cnt = bucket(src, dst, w)
    (agg1f,) = agg_deg(pk_list, w_list, cnt, x)
    agg1f = agg1f.reshape(npad, stride1)
    agg1 = agg1f[:N, :D]
    deg2d = agg1f[:N, D].reshape(N, 1)
    h1 = _dense_layer(x, agg1, deg2d, W_self0, W_neigh0,
                      b0.reshape(1, D), relu=True)
    (agg2,) = agg(pk_list, w_list, cnt, h1)
    agg2 = agg2.reshape(npad, stride2)[:N, :D]
    out = _dense_layer(h1, agg2, deg2d, W_self1, W_neigh1,
                       b1.reshape(1, D), relu=False)
    return out


# speedup vs baseline: 4.5204x; 1.1093x over previous
import jax
import jax.numpy as jnp
from jax import lax
from jax.experimental import pallas as pl
from jax.experimental.pallas import tpu as pltpu
from jax.experimental.pallas import tpu_sc as plsc

N = 10000
E = 160000
D = 256

NC = 2
NS = 16
NT = NC * NS
L = 16

BATCH = 64
NBG = BATCH // L
LC = 2048
PKS = 512

_SC_PARAMS = pltpu.CompilerParams(needs_layout_passes=False)


def _mesh():
    return plsc.VectorSubcoreMesh(core_axis_name="c", subcore_axis_name="s",
                                  num_cores=NC, num_subcores=NS)


def _derived(n, e):
    rpt = ((n + NT * 8 - 1) // (NT * 8)) * 8
    region = ((e + LC + 127) // 128) * 128 + 128
    return rpt, region


def _make_bucket(n, e, ce, flush):
    rpt, region = _derived(n, e)
    assert rpt < PKS and e % ce == 0 and ce % L == 0 and ce % 8 == 0
    nch = e // ce
    ngrp = ce // L
    stash = ((ce + flush + L + 7) // 8) * 8
    stsz = stash + L

    assert nch % 2 == 0

    def body(src_hbm, dst_hbm, w_hbm, pk_out, w_out, cnt_out,
             src_ck0, dst_ck0, w_ck0, sem0,
             src_ck1, dst_ck1, w_ck1, sem1, st_pk, st_w, cnts, offs, cntb):
        csets = ((src_ck0, dst_ck0, w_ck0, sem0),
                 (src_ck1, dst_ck1, w_ck1, sem1))
        c = lax.axis_index("c")
        s = lax.axis_index("s")
        tid = c * NS + s
        lo = tid * rpt
        tbase = tid * region
        iota = lax.iota(jnp.int32, L)

        def start_load(ch, cset):
            ebase = ch * ce
            pltpu.async_copy(src_hbm.at[pl.ds(ebase, ce)], cset[0], cset[3])
            pltpu.async_copy(dst_hbm.at[pl.ds(ebase, ce)], cset[1], cset[3])
            pltpu.async_copy(w_hbm.at[pl.ds(ebase, ce)], cset[2], cset[3])

        def wait_load(ch, cset):
            ebase = ch * ce
            pltpu.make_async_copy(
                src_hbm.at[pl.ds(ebase, ce)], cset[0], cset[3]).wait()
            pltpu.make_async_copy(
                dst_hbm.at[pl.ds(ebase, ce)], cset[1], cset[3]).wait()
            pltpu.make_async_copy(
                w_hbm.at[pl.ds(ebase, ce)], cset[2], cset[3]).wait()

        start_load(0, csets[0])

        def chunk(ch, carry, src_ck, dst_ck, w_ck):
            ptr, total = carry
            ngrp16 = (ngrp + L - 1) // L
            cnts[pl.ds((ngrp16 - 1) * L, L)] = jnp.zeros((L,), jnp.int32)

            @plsc.parallel_loop(0, ngrp, 1, unroll=4)
            def _count(g):
                d16 = dst_ck[pl.ds(g * L, L)]
                dl = d16 - lo
                m = (dl >= 0) & (dl < rpt)
                cnt = plsc.all_reduce_population_count(m)
                plsc.store_scatter(cnts, [jnp.full((L,), g, jnp.int32)], cnt)

            def pre(i, p):
                c16 = cnts[pl.ds(i * L, L)]
                cs = plsc.cumsum(c16)
                offs[pl.ds(i * L, L)] = p + cs - c16
                return p + cs[L - 1]

            new_ptr = lax.fori_loop(0, ngrp16, pre, ptr)

            @plsc.parallel_loop(0, ngrp, 1, unroll=4)
            def _scatter(g):
                d16 = dst_ck[pl.ds(g * L, L)]
                s16 = src_ck[pl.ds(g * L, L)]
                w16 = w_ck[pl.ds(g * L, L)]
                dl = d16 - lo
                m = (dl >= 0) & (dl < rpt)
                csum = plsc.cumsum(jnp.where(m, 1, 0))
                off = plsc.load_gather(offs, [jnp.full((L,), g, jnp.int32)])
                pos = jnp.where(m, off + csum - 1, stash + iota)
                plsc.store_scatter(st_pk, [pos], s16 * PKS + dl)
                plsc.store_scatter(st_w, [pos], w16)

            ptr = new_ptr
            nfl = ptr // flush

            def fl(f, _):
                o = f * flush
                dst_off = pl.multiple_of(tbase + total + o, flush)
                pltpu.sync_copy(st_pk.at[pl.ds(o, flush)],
                                pk_out.at[pl.ds(dst_off, flush)])
                pltpu.sync_copy(st_w.at[pl.ds(o, flush)],
                                w_out.at[pl.ds(dst_off, flush)])
                return 0

            lax.fori_loop(0, nfl, fl, 0)
            moved = nfl * flush

            @pl.when(nfl > 0)
            def _tail():
                for g in range(flush // L):
                    sl = pl.ds(g * L, L)
                    st_pk[sl] = st_pk[pl.ds(moved + g * L, L)]
                    st_w[sl] = st_w[pl.ds(moved + g * L, L)]

            return ptr - moved, total + moved

        def pair(p, carry):
            for sub in range(2):
                ch = 2 * p + sub
                cur = csets[sub]
                wait_load(ch, cur)

                @pl.when(ch + 1 < nch)
                def _():
                    start_load(ch + 1, csets[1 - sub])

                carry = chunk(ch, carry, cur[0], cur[1], cur[2])
            return carry

        ptr, total = lax.fori_loop(0, nch // 2, pair,
                                   (jnp.int32(0), jnp.int32(0)))

        nfin = (ptr + 127) // 128

        def ffin(f, _):
            o = f * 128
            dst_off = pl.multiple_of(tbase + total + o, 128)
            pltpu.sync_copy(st_pk.at[pl.ds(o, 128)],
                            pk_out.at[pl.ds(dst_off, 128)])
            pltpu.sync_copy(st_w.at[pl.ds(o, 128)],
                            w_out.at[pl.ds(dst_off, 128)])
            return 0

        lax.fori_loop(0, nfin, ffin, 0)
        cntb[pl.ds(0, L)] = jnp.full((L,), total + ptr, jnp.int32)
        pltpu.sync_copy(cntb, cnt_out.at[pl.ds(tid * L, L)])

    out_type = (
        jax.ShapeDtypeStruct((NT * region,), jnp.int32),
        jax.ShapeDtypeStruct((NT * region,), jnp.float32),
        jax.ShapeDtypeStruct((NT * L,), jnp.int32),
    )
    scratch = []
    for _ in range(2):
        scratch += [
            pltpu.VMEM((ce,), jnp.int32),
            pltpu.VMEM((ce,), jnp.int32),
            pltpu.VMEM((ce,), jnp.float32),
            pltpu.SemaphoreType.DMA,
        ]
    ngrp16 = (ngrp + L - 1) // L
    scratch += [
        pltpu.VMEM((stsz,), jnp.int32),
        pltpu.VMEM((stsz,), jnp.float32),
        pltpu.VMEM((ngrp16 * L,), jnp.int32),
        pltpu.VMEM((ngrp16 * L,), jnp.int32),
        pltpu.VMEM((L,), jnp.int32),
    ]
    return pl.kernel(body, out_type=out_type, mesh=_mesh(),
                     scratch_types=scratch,
                     compiler_params=_SC_PARAMS)


def _make_agg(n, e, d, compute_deg):
    assert d % L == 0
    nd16 = d // L
    rpt, region = _derived(n, e)
    npad = NT * rpt
    trash = rpt
    stride = d + L if compute_deg else d
    accw = (rpt + 1) * stride

    bpc = LC // BATCH

    def body(pk_hbm, wl_hbm, cnt_hbm, h_hbm, *rest):
        (out_hbm, acc, lst_pk, lst_w,
         b_src0, b_dl0, b_w0, gbuf0, sem0, sem0b,
         b_src1, b_dl1, b_w1, gbuf1, sem1, sem1b, cntb) = rest
        bufs = ((b_src0, b_dl0, b_w0, gbuf0, sem0, sem0b),
                (b_src1, b_dl1, b_w1, gbuf1, sem1, sem1b))
        c = lax.axis_index("c")
        s = lax.axis_index("s")
        tid = c * NS + s
        tbase = tid * region
        iota = lax.iota(jnp.int32, L)
        zv = jnp.zeros((L,), jnp.float32)
        e0 = jnp.where(iota == 0, 1.0, 0.0)

        def zrow(r, _):
            acc[pl.ds(r * L, L)] = zv
            return 0
        lax.fori_loop(0, accw // L, zrow, 0)

        pltpu.sync_copy(cnt_hbm.at[pl.ds(tid * L, L)], cntb)
        cnt = cntb[pl.ds(0, L)][0]
        nb = (cnt + (BATCH - 1)) // BATCH
        ncl = (nb + (bpc - 1)) // (bpc)

        half = BATCH // 2

        def start_gather(bset):
            b_src, gbuf, sem, sem2 = bset[0], bset[3], bset[4], bset[5]
            pltpu.async_copy(h_hbm.at[b_src.at[pl.ds(0, half)]],
                             gbuf.at[pl.ds(0, half)], sem)
            pltpu.async_copy(h_hbm.at[b_src.at[pl.ds(half, half)]],
                             gbuf.at[pl.ds(half, half)], sem2)

        def wait_gather(bset):
            b_src, gbuf, sem, sem2 = bset[0], bset[3], bset[4], bset[5]
            pltpu.make_async_copy(h_hbm.at[b_src.at[pl.ds(0, half)]],
                                  gbuf.at[pl.ds(0, half)], sem).wait()
            pltpu.make_async_copy(h_hbm.at[b_src.at[pl.ds(half, half)]],
                                  gbuf.at[pl.ds(half, half)], sem2).wait()

        def prep(ci, bi, b_src, b_dl, b_w):
            base = ci * bpc + bi
            for g in range(NBG):
                off = bi * BATCH + g * L
                valid = (iota + (base * BATCH + g * L)) < cnt
                pk = lst_pk[pl.ds(off, L)]
                wv = lst_w[pl.ds(off, L)]
                sv = lax.shift_right_logical(pk, 9)
                dv = lax.bitwise_and(pk, PKS - 1)
                sl = pl.ds(g * L, L)
                b_src[sl] = jnp.where(valid, sv, 0)
                b_dl[sl] = jnp.where(valid, dv, trash)
                b_w[sl] = jnp.where(valid, wv, 0.0)

        def compute(b_dl, b_w, gbuf):
            @plsc.parallel_loop(0, BATCH, 1, unroll=4)
            def _edges(r):
                rr = jnp.full((L,), r, jnp.int32)
                wsp = plsc.load_gather(b_w, [rr])
                dlv = plsc.load_gather(b_dl, [rr])
                base = dlv * stride + iota
                for k in range(nd16):
                    plsc.addupdate_scatter(
                        acc, [base + (k * L)],
                        gbuf[r, pl.ds(k * L, L)] * wsp)
                if compute_deg:
                    plsc.addupdate_scatter(acc, [base + d], e0)

        def list_chunk(ci, _):
            pltpu.sync_copy(pk_hbm.at[pl.ds(tbase + ci * LC, LC)], lst_pk)
            pltpu.sync_copy(wl_hbm.at[pl.ds(tbase + ci * LC, LC)], lst_w)
            nbi = jnp.minimum(nb - ci * bpc, bpc)

            @pl.when(nbi > 0)
            def _prime():
                prep(ci, jnp.int32(0), b_src0, b_dl0, b_w0)
                start_gather(bufs[0])

            def pair(p, _):
                for sub in range(2):
                    bi = 2 * p + sub
                    cur = bufs[sub]
                    nxt = bufs[1 - sub]

                    @pl.when(bi < nbi)
                    def _():
                        wait_gather(cur)

                        @pl.when(bi + 1 < nbi)
                        def _():
                            prep(ci, bi + 1, nxt[0], nxt[1], nxt[2])
                            start_gather(nxt)

                        compute(cur[1], cur[2], cur[3])
                return 0

            lax.fori_loop(0, (nbi + 1) // 2, pair, 0)
            return 0

        lax.fori_loop(0, ncl, list_chunk, 0)

        pltpu.sync_copy(acc.at[pl.ds(0, rpt * stride)],
                        out_hbm.at[pl.ds(tid * rpt * stride, rpt * stride)])

    out_type = [jax.ShapeDtypeStruct((npad * stride,), jnp.float32)]
    scratch = [pltpu.VMEM((accw,), jnp.float32)]
    scratch += [
        pltpu.VMEM((LC,), jnp.int32),
        pltpu.VMEM((LC,), jnp.float32),
    ]
    for _ in range(2):
        scratch += [
            pltpu.VMEM((BATCH,), jnp.int32),
            pltpu.VMEM((BATCH,), jnp.int32),
            pltpu.VMEM((BATCH,), jnp.float32),
            pltpu.VMEM((BATCH, d), jnp.float32),
            pltpu.SemaphoreType.DMA,
            pltpu.SemaphoreType.DMA,
        ]
    scratch += [pltpu.VMEM((L,), jnp.int32)]
    return pl.kernel(body, out_type=tuple(out_type), mesh=_mesh(),
                     scratch_types=scratch,
                     compiler_params=_SC_PARAMS), npad, stride


def _dense_layer(h, agg_raw, deg, w_self, w_neigh, b, relu, bm=400):
    n, d = h.shape
    assert n % bm == 0
    dn = (((1,), (1,)), ((), ()))

    def body(x_ref, agg_ref, deg_ref, ws_ref, wn_ref, b_ref, o_ref):
        inv = 1.0 / jnp.maximum(deg_ref[...], 1.0)
        scaled = agg_ref[...] * inv
        acc = lax.dot_general(x_ref[...], ws_ref[...], dn,
                              preferred_element_type=jnp.float32)
        acc = acc + lax.dot_general(scaled, wn_ref[...], dn,
                                    preferred_element_type=jnp.float32)
        acc = acc + b_ref[...]
        o_ref[...] = jnp.maximum(acc, 0.0) if relu else acc

    return pl.pallas_call(
        body,
        grid=(n // bm,),
        in_specs=[
            pl.BlockSpec((bm, d), lambda i: (i, 0)),
            pl.BlockSpec((bm, d), lambda i: (i, 0)),
            pl.BlockSpec((bm, 1), lambda i: (i, 0)),
            pl.BlockSpec((d, d), lambda i: (0, 0)),
            pl.BlockSpec((d, d), lambda i: (0, 0)),
            pl.BlockSpec((1, d), lambda i: (0, 0)),
        ],
        out_specs=pl.BlockSpec((bm, d), lambda i: (i, 0)),
        out_shape=jax.ShapeDtypeStruct((n, d), jnp.float32),
    )(h, agg_raw, deg, w_self, w_neigh, b)


def kernel(x, edge_index, edge_weight, W_self0, W_neigh0, b0,
           W_self1, W_neigh1, b1):
    src = edge_index[0].astype(jnp.int32)
    dst = edge_index[1].astype(jnp.int32)
    w = edge_weight.astype(jnp.float32)

    bucket = _make_bucket(N, E, ce=4000, flush=1024)
    agg_deg, npad, stride1 = _make_agg(N, E, D, compute_deg=True)
    agg, _, stride2 = _make_agg(N, E, D, compute_deg=False)

    pk_list, w_list, cnt = bucket(src, dst, w)
    (agg1f,) = agg_deg(pk_list, w_list, cnt, x)
    agg1f = agg1f.reshape(npad, stride1)
    agg1 = agg1f[:N, :D]
    deg2d = agg1f[:N, D].reshape(N, 1)
    h1 = _dense_layer(x, agg1, deg2d, W_self0, W_neigh0,
                      b0.reshape(1, D), relu=True)
    (agg2,) = agg(pk_list, w_list, cnt, h1)
    agg2 = agg2.reshape(npad, stride2)[:N, :D]
    out = _dense_layer(h1, agg2, deg2d, W_self1, W_neigh1,
                       b1.reshape(1, D), relu=False)
    return out


# speedup vs baseline: 4.5600x; 1.0087x over previous
import jax
import jax.numpy as jnp
from jax import lax
from jax.experimental import pallas as pl
from jax.experimental.pallas import tpu as pltpu
from jax.experimental.pallas import tpu_sc as plsc

N = 10000
E = 160000
D = 256

NC = 2
NS = 16
NT = NC * NS
L = 16

BATCH = 64
NBG = BATCH // L
LC = 2048
PKS = 512

_SC_PARAMS = pltpu.CompilerParams(needs_layout_passes=False)


def _mesh():
    return plsc.VectorSubcoreMesh(core_axis_name="c", subcore_axis_name="s",
                                  num_cores=NC, num_subcores=NS)


def _derived(n, e):
    rpt = ((n + NT * 8 - 1) // (NT * 8)) * 8
    region = ((e + LC + 127) // 128) * 128 + 128
    return rpt, region


def _make_bucket(n, e, ce, flush):
    rpt, region = _derived(n, e)
    assert rpt < PKS and e % ce == 0 and ce % L == 0 and ce % 8 == 0
    nch = e // ce
    ngrp = ce // L
    stash = ((ce + flush + L + 7) // 8) * 8
    stsz = stash + L

    assert nch % 2 == 0

    def body(src_hbm, dst_hbm, w_hbm, pk_out, w_out, cnt_out,
             src_ck0, dst_ck0, w_ck0, sem0,
             src_ck1, dst_ck1, w_ck1, sem1, st_pk, st_w, cnts, offs, cntb):
        csets = ((src_ck0, dst_ck0, w_ck0, sem0),
                 (src_ck1, dst_ck1, w_ck1, sem1))
        c = lax.axis_index("c")
        s = lax.axis_index("s")
        tid = c * NS + s
        lo = tid * rpt
        tbase = tid * region
        iota = lax.iota(jnp.int32, L)

        def start_load(ch, cset):
            ebase = ch * ce
            pltpu.async_copy(src_hbm.at[pl.ds(ebase, ce)], cset[0], cset[3])
            pltpu.async_copy(dst_hbm.at[pl.ds(ebase, ce)], cset[1], cset[3])
            pltpu.async_copy(w_hbm.at[pl.ds(ebase, ce)], cset[2], cset[3])

        def wait_load(ch, cset):
            ebase = ch * ce
            pltpu.make_async_copy(
                src_hbm.at[pl.ds(ebase, ce)], cset[0], cset[3]).wait()
            pltpu.make_async_copy(
                dst_hbm.at[pl.ds(ebase, ce)], cset[1], cset[3]).wait()
            pltpu.make_async_copy(
                w_hbm.at[pl.ds(ebase, ce)], cset[2], cset[3]).wait()

        start_load(0, csets[0])

        def chunk(ch, carry, src_ck, dst_ck, w_ck):
            ptr, total = carry
            ngrp16 = (ngrp + L - 1) // L
            cnts[pl.ds((ngrp16 - 1) * L, L)] = jnp.zeros((L,), jnp.int32)

            @plsc.parallel_loop(0, ngrp, 1, unroll=4)
            def _count(g):
                d16 = dst_ck[pl.ds(g * L, L)]
                dl = d16 - lo
                m = (dl >= 0) & (dl < rpt)
                cnt = plsc.all_reduce_population_count(m)
                plsc.store_scatter(cnts, [jnp.full((L,), g, jnp.int32)], cnt)

            def pre(i, p):
                c16 = cnts[pl.ds(i * L, L)]
                cs = plsc.cumsum(c16)
                offs[pl.ds(i * L, L)] = p + cs - c16
                return p + cs[L - 1]

            new_ptr = lax.fori_loop(0, ngrp16, pre, ptr)

            @plsc.parallel_loop(0, ngrp, 1, unroll=4)
            def _scatter(g):
                d16 = dst_ck[pl.ds(g * L, L)]
                s16 = src_ck[pl.ds(g * L, L)]
                w16 = w_ck[pl.ds(g * L, L)]
                dl = d16 - lo
                m = (dl >= 0) & (dl < rpt)
                csum = plsc.cumsum(jnp.where(m, 1, 0))
                off = plsc.load_gather(offs, [jnp.full((L,), g, jnp.int32)])
                pos = jnp.where(m, off + csum - 1, stash + iota)
                plsc.store_scatter(st_pk, [pos], s16 * PKS + dl)
                plsc.store_scatter(st_w, [pos], w16)

            ptr = new_ptr
            nfl = ptr // flush

            def fl(f, _):
                o = f * flush
                dst_off = pl.multiple_of(tbase + total + o, flush)
                pltpu.sync_copy(st_pk.at[pl.ds(o, flush)],
                                pk_out.at[pl.ds(dst_off, flush)])
                pltpu.sync_copy(st_w.at[pl.ds(o, flush)],
                                w_out.at[pl.ds(dst_off, flush)])
                return 0

            lax.fori_loop(0, nfl, fl, 0)
            moved = nfl * flush

            @pl.when(nfl > 0)
            def _tail():
                for g in range(flush // L):
                    sl = pl.ds(g * L, L)
                    st_pk[sl] = st_pk[pl.ds(moved + g * L, L)]
                    st_w[sl] = st_w[pl.ds(moved + g * L, L)]

            return ptr - moved, total + moved

        def pair(p, carry):
            for sub in range(2):
                ch = 2 * p + sub
                cur = csets[sub]
                wait_load(ch, cur)

                @pl.when(ch + 1 < nch)
                def _():
                    start_load(ch + 1, csets[1 - sub])

                carry = chunk(ch, carry, cur[0], cur[1], cur[2])
            return carry

        ptr, total = lax.fori_loop(0, nch // 2, pair,
                                   (jnp.int32(0), jnp.int32(0)))

        nfin = (ptr + 127) // 128

        def ffin(f, _):
            o = f * 128
            dst_off = pl.multiple_of(tbase + total + o, 128)
            pltpu.sync_copy(st_pk.at[pl.ds(o, 128)],
                            pk_out.at[pl.ds(dst_off, 128)])
            pltpu.sync_copy(st_w.at[pl.ds(o, 128)],
                            w_out.at[pl.ds(dst_off, 128)])
            return 0

        lax.fori_loop(0, nfin, ffin, 0)
        cntb[pl.ds(0, L)] = jnp.full((L,), total + ptr, jnp.int32)
        pltpu.sync_copy(cntb, cnt_out.at[pl.ds(tid * L, L)])

    out_type = (
        jax.ShapeDtypeStruct((NT * region,), jnp.int32),
        jax.ShapeDtypeStruct((NT * region,), jnp.float32),
        jax.ShapeDtypeStruct((NT * L,), jnp.int32),
    )
    scratch = []
    for _ in range(2):
        scratch += [
            pltpu.VMEM((ce,), jnp.int32),
            pltpu.VMEM((ce,), jnp.int32),
            pltpu.VMEM((ce,), jnp.float32),
            pltpu.SemaphoreType.DMA,
        ]
    ngrp16 = (ngrp + L - 1) // L
    scratch += [
        pltpu.VMEM((stsz,), jnp.int32),
        pltpu.VMEM((stsz,), jnp.float32),
        pltpu.VMEM((ngrp16 * L,), jnp.int32),
        pltpu.VMEM((ngrp16 * L,), jnp.int32),
        pltpu.VMEM((L,), jnp.int32),
    ]
    return pl.kernel(body, out_type=out_type, mesh=_mesh(),
                     scratch_types=scratch,
                     compiler_params=_SC_PARAMS)


def _make_agg(n, e, d, compute_deg):
    assert d % L == 0
    nd16 = d // L
    rpt, region = _derived(n, e)
    npad = NT * rpt
    trash = rpt
    stride = d + L if compute_deg else d
    accw = (rpt + 1) * stride

    bpc = LC // BATCH

    def body(pk_hbm, wl_hbm, cnt_hbm, h_hbm, *rest):
        (out_hbm, acc, lst_pk, lst_w,
         b_src0, b_dl0, b_w0, gbuf0, sem0, sem0b,
         b_src1, b_dl1, b_w1, gbuf1, sem1, sem1b, cntb) = rest
        bufs = ((b_src0, b_dl0, b_w0, gbuf0, sem0, sem0b),
                (b_src1, b_dl1, b_w1, gbuf1, sem1, sem1b))
        c = lax.axis_index("c")
        s = lax.axis_index("s")
        tid = c * NS + s
        tbase = tid * region
        iota = lax.iota(jnp.int32, L)
        zv = jnp.zeros((L,), jnp.float32)
        e0 = jnp.where(iota == 0, 1.0, 0.0)

        def zrow(r, _):
            acc[pl.ds(r * L, L)] = zv
            return 0
        lax.fori_loop(0, accw // L, zrow, 0)

        pltpu.sync_copy(cnt_hbm.at[pl.ds(tid * L, L)], cntb)
        cnt = cntb[pl.ds(0, L)][0]
        nb = (cnt + (BATCH - 1)) // BATCH
        ncl = (nb + (bpc - 1)) // (bpc)

        half = BATCH // 2

        def start_gather(bset):
            b_src, gbuf, sem, sem2 = bset[0], bset[3], bset[4], bset[5]
            pltpu.async_copy(h_hbm.at[b_src.at[pl.ds(0, half)]],
                             gbuf.at[pl.ds(0, half)], sem)
            pltpu.async_copy(h_hbm.at[b_src.at[pl.ds(half, half)]],
                             gbuf.at[pl.ds(half, half)], sem2)

        def wait_gather(bset):
            b_src, gbuf, sem, sem2 = bset[0], bset[3], bset[4], bset[5]
            pltpu.make_async_copy(h_hbm.at[b_src.at[pl.ds(0, half)]],
                                  gbuf.at[pl.ds(0, half)], sem).wait()
            pltpu.make_async_copy(h_hbm.at[b_src.at[pl.ds(half, half)]],
                                  gbuf.at[pl.ds(half, half)], sem2).wait()

        def prep(ci, bi, b_src, b_dl, b_w):
            base = ci * bpc + bi
            for g in range(NBG):
                off = bi * BATCH + g * L
                valid = (iota + (base * BATCH + g * L)) < cnt
                pk = lst_pk[pl.ds(off, L)]
                wv = lst_w[pl.ds(off, L)]
                sv = lax.shift_right_logical(pk, 9)
                dv = lax.bitwise_and(pk, PKS - 1)
                sl = pl.ds(g * L, L)
                b_src[sl] = jnp.where(valid, sv, 0)
                b_dl[sl] = jnp.where(valid, dv, trash)
                b_w[sl] = jnp.where(valid, wv, 0.0)

        def compute(b_dl, b_w, gbuf):
            @plsc.parallel_loop(0, BATCH, 1, unroll=4)
            def _edges(r):
                rr = jnp.full((L,), r, jnp.int32)
                wsp = plsc.load_gather(b_w, [rr])
                dlv = plsc.load_gather(b_dl, [rr])
                dbase = dlv[0] * stride
                for k in range(nd16):
                    plsc.addupdate(acc.at[pl.ds(dbase + k * L, L)],
                                   gbuf[r, pl.ds(k * L, L)] * wsp)
                if compute_deg:
                    plsc.addupdate(acc.at[pl.ds(dbase + d, L)], e0)

        def list_chunk(ci, _):
            pltpu.sync_copy(pk_hbm.at[pl.ds(tbase + ci * LC, LC)], lst_pk)
            pltpu.sync_copy(wl_hbm.at[pl.ds(tbase + ci * LC, LC)], lst_w)
            nbi = jnp.minimum(nb - ci * bpc, bpc)

            @pl.when(nbi > 0)
            def _prime():
                prep(ci, jnp.int32(0), b_src0, b_dl0, b_w0)
                start_gather(bufs[0])

            def pair(p, _):
                for sub in range(2):
                    bi = 2 * p + sub
                    cur = bufs[sub]
                    nxt = bufs[1 - sub]

                    @pl.when(bi < nbi)
                    def _():
                        wait_gather(cur)

                        @pl.when(bi + 1 < nbi)
                        def _():
                            prep(ci, bi + 1, nxt[0], nxt[1], nxt[2])
                            start_gather(nxt)

                        compute(cur[1], cur[2], cur[3])
                return 0

            lax.fori_loop(0, (nbi + 1) // 2, pair, 0)
            return 0

        lax.fori_loop(0, ncl, list_chunk, 0)

        pltpu.sync_copy(acc.at[pl.ds(0, rpt * stride)],
                        out_hbm.at[pl.ds(tid * rpt * stride, rpt * stride)])

    out_type = [jax.ShapeDtypeStruct((npad * stride,), jnp.float32)]
    scratch = [pltpu.VMEM((accw,), jnp.float32)]
    scratch += [
        pltpu.VMEM((LC,), jnp.int32),
        pltpu.VMEM((LC,), jnp.float32),
    ]
    for _ in range(2):
        scratch += [
            pltpu.VMEM((BATCH,), jnp.int32),
            pltpu.VMEM((BATCH,), jnp.int32),
            pltpu.VMEM((BATCH,), jnp.float32),
            pltpu.VMEM((BATCH, d), jnp.float32),
            pltpu.SemaphoreType.DMA,
            pltpu.SemaphoreType.DMA,
        ]
    scratch += [pltpu.VMEM((L,), jnp.int32)]
    return pl.kernel(body, out_type=tuple(out_type), mesh=_mesh(),
                     scratch_types=scratch,
                     compiler_params=_SC_PARAMS), npad, stride


def _dense_layer(h, agg_raw, deg, w_self, w_neigh, b, relu, bm=400):
    n, d = h.shape
    assert n % bm == 0
    dn = (((1,), (1,)), ((), ()))

    def body(x_ref, agg_ref, deg_ref, ws_ref, wn_ref, b_ref, o_ref):
        inv = 1.0 / jnp.maximum(deg_ref[...], 1.0)
        scaled = agg_ref[...] * inv
        acc = lax.dot_general(x_ref[...], ws_ref[...], dn,
                              preferred_element_type=jnp.float32)
        acc = acc + lax.dot_general(scaled, wn_ref[...], dn,
                                    preferred_element_type=jnp.float32)
        acc = acc + b_ref[...]
        o_ref[...] = jnp.maximum(acc, 0.0) if relu else acc

    return pl.pallas_call(
        body,
        grid=(n // bm,),
        in_specs=[
            pl.BlockSpec((bm, d), lambda i: (i, 0)),
            pl.BlockSpec((bm, d), lambda i: (i, 0)),
            pl.BlockSpec((bm, 1), lambda i: (i, 0)),
            pl.BlockSpec((d, d), lambda i: (0, 0)),
            pl.BlockSpec((d, d), lambda i: (0, 0)),
            pl.BlockSpec((1, d), lambda i: (0, 0)),
        ],
        out_specs=pl.BlockSpec((bm, d), lambda i: (i, 0)),
        out_shape=jax.ShapeDtypeStruct((n, d), jnp.float32),
    )(h, agg_raw, deg, w_self, w_neigh, b)


def kernel(x, edge_index, edge_weight, W_self0, W_neigh0, b0,
           W_self1, W_neigh1, b1):
    src = edge_index[0].astype(jnp.int32)
    dst = edge_index[1].astype(jnp.int32)
    w = edge_weight.astype(jnp.float32)

    bucket = _make_bucket(N, E, ce=4000, flush=1024)
    agg_deg, npad, stride1 = _make_agg(N, E, D, compute_deg=True)
    agg, _, stride2 = _make_agg(N, E, D, compute_deg=False)

    pk_list, w_list, cnt = bucket(src, dst, w)
    (agg1f,) = agg_deg(pk_list, w_list, cnt, x)
    agg1f = agg1f.reshape(npad, stride1)
    agg1 = agg1f[:N, :D]
    deg2d = agg1f[:N, D].reshape(N, 1)
    h1 = _dense_layer(x, agg1, deg2d, W_self0, W_neigh0,
                      b0.reshape(1, D), relu=True)
    (agg2,) = agg(pk_list, w_list, cnt, h1)
    agg2 = agg2.reshape(npad, stride2)[:N, :D]
    out = _dense_layer(h1, agg2, deg2d, W_self1, W_neigh1,
                       b1.reshape(1, D), relu=False)
    return out
